# baseline (speedup 1.0000x reference)
import functools

import jax
import jax.numpy as jnp
from jax import lax
from jax.experimental import pallas as pl
from jax.experimental.pallas import tpu as pltpu
from jax.experimental.pallas import tpu_sc as plsc

H = 64
B = 1024
E_ITEM = 51200
E_SOC = 20480
NC = 2
NS = 16
NW = NC * NS
GC = 80
YW = 2 * H


def _mesh():
    return plsc.VectorSubcoreMesh(core_axis_name="c", subcore_axis_name="s")


_SC_PARAMS = pltpu.CompilerParams(use_tc_tiling_on_sc=False,
                                  needs_layout_passes=True)


def _worker_id():
    return lax.axis_index("s") * NC + lax.axis_index("c")


def _gather_task(table, idx, out, n, idx_v, rows_v, sem, w):
    per = n // NW
    base = pl.multiple_of(w * per, 8)
    if per >= GC:
        def body(i, carry):
            off = pl.multiple_of(base + i * GC, 8)
            pltpu.sync_copy(idx.at[pl.ds(off, GC)], idx_v)
            pltpu.async_copy(table.at[idx_v], rows_v, sem).wait()
            pltpu.sync_copy(rows_v, out.at[pl.ds(off, GC)])
            return carry
        lax.fori_loop(0, per // GC, body, 0)
    else:
        sl = pl.ds(0, per)
        pltpu.sync_copy(idx.at[pl.ds(base, per)], idx_v.at[sl])
        pltpu.async_copy(table.at[idx_v.at[sl]], rows_v.at[sl], sem).wait()
        pltpu.sync_copy(rows_v.at[sl], out.at[pl.ds(base, per)])


def _sc_gather_stage1(user_table, item_table, user, batch_target, item,
                      social, batch_i_users):
    outs = [
        jax.ShapeDtypeStruct((B, H), jnp.float32),
        jax.ShapeDtypeStruct((B, H), jnp.float32),
        jax.ShapeDtypeStruct((E_ITEM, H), jnp.float32),
        jax.ShapeDtypeStruct((E_SOC, H), jnp.float32),
        jax.ShapeDtypeStruct((E_ITEM, H), jnp.float32),
    ]

    @functools.partial(
        pl.kernel, out_type=outs, mesh=_mesh(),
        compiler_params=_SC_PARAMS,
        scratch_types=[pltpu.VMEM((GC,), jnp.int32),
                       pltpu.VMEM((GC, H), jnp.float32),
                       pltpu.SemaphoreType.DMA])
    def k(user_t, item_t, user_i, btgt_i, item_i, social_i, biu_i,
          user_emb, tgt, item_emb, social_emb, item_u_emb,
          idx_v, rows_v, sem):
        w = _worker_id()
        _gather_task(user_t, user_i, user_emb, B, idx_v, rows_v, sem, w)
        _gather_task(item_t, btgt_i, tgt, B, idx_v, rows_v, sem, w)
        _gather_task(item_t, item_i, item_emb, E_ITEM, idx_v, rows_v, sem, w)
        _gather_task(user_t, social_i, social_emb, E_SOC, idx_v, rows_v, sem, w)
        _gather_task(user_t, biu_i, item_u_emb, E_ITEM, idx_v, rows_v, sem, w)

    return k(user_table, item_table, user, batch_target, item, social,
             batch_i_users)


def _sc_gather_stage2(user_emb, tgt, item4user, social4user, batch_u_item):
    outs = [
        jax.ShapeDtypeStruct((E_ITEM, H), jnp.float32),
        jax.ShapeDtypeStruct((E_SOC, H), jnp.float32),
        jax.ShapeDtypeStruct((E_ITEM, H), jnp.float32),
    ]

    @functools.partial(
        pl.kernel, out_type=outs, mesh=_mesh(),
        compiler_params=_SC_PARAMS,
        scratch_types=[pltpu.VMEM((GC,), jnp.int32),
                       pltpu.VMEM((GC, H), jnp.float32),
                       pltpu.SemaphoreType.DMA])
    def k(ue_t, tgt_t, i4u_i, s4u_i, bui_i, ue_g, su, tgt_g,
          idx_v, rows_v, sem):
        w = _worker_id()
        _gather_task(ue_t, i4u_i, ue_g, E_ITEM, idx_v, rows_v, sem, w)
        _gather_task(ue_t, s4u_i, su, E_SOC, idx_v, rows_v, sem, w)
        _gather_task(tgt_t, bui_i, tgt_g, E_ITEM, idx_v, rows_v, sem, w)

    return k(user_emb, tgt, item4user, social4user, batch_u_item)


CE = 2048


def _tc_edges_proj(x1, rids3, xg, r_table, w_pre, b_pre, w0, b0, w1, b1, n):
    grid = n // CE

    def body(x1_ref, rid_ref, xg_ref, rt_ref, wpre_ref, bpre_ref,
             w0_ref, b0_ref, w1_ref, b1_ref, y_ref):
        ids = rid_ref[0, 0, :]
        oh = (ids[:, None] == lax.broadcasted_iota(jnp.int32, (CE, 16), 1)
              ).astype(jnp.float32)
        x2 = oh @ rt_ref[...]
        wpre = wpre_ref[...]
        xia = (x1_ref[...] @ wpre[:, :H].T + x2 @ wpre[:, H:].T
               + bpre_ref[...])
        w0m = w0_ref[...]
        a1 = jnp.maximum(
            xia @ w0m[:, :H].T + xg_ref[...] @ w0m[:, H:].T + b0_ref[...], 0.0)
        l = a1 @ w1_ref[...].T + b1_ref[...]
        e = jnp.exp(l)
        y_ref[...] = jnp.concatenate([xia * e, e], axis=1)

    rt16 = jnp.zeros((16, H), jnp.float32).at[:10].set(r_table)
    return pl.pallas_call(
        body,
        grid=(grid,),
        in_specs=[
            pl.BlockSpec((CE, H), lambda i: (i, 0)),
            pl.BlockSpec((1, 1, CE), lambda i: (i, 0, 0)),
            pl.BlockSpec((CE, H), lambda i: (i, 0)),
            pl.BlockSpec((16, H), lambda i: (0, 0)),
            pl.BlockSpec((H, 2 * H), lambda i: (0, 0)),
            pl.BlockSpec((1, H), lambda i: (0, 0)),
            pl.BlockSpec((H, 2 * H), lambda i: (0, 0)),
            pl.BlockSpec((1, H), lambda i: (0, 0)),
            pl.BlockSpec((H, H), lambda i: (0, 0)),
            pl.BlockSpec((1, 1), lambda i: (0, 0)),
        ],
        out_specs=pl.BlockSpec((CE, YW), lambda i: (i, 0)),
        out_shape=jax.ShapeDtypeStruct((n, YW), jnp.float32),
    )(x1, rids3, xg, rt16, w_pre, b_pre, w0, b0, w1, b1)


def _tc_edges_soc(se, su, w0, b0, w1, b1):
    grid = E_SOC // CE

    def body(se_ref, su_ref, w0_ref, b0_ref, w1_ref, b1_ref, y_ref):
        su_x = su_ref[...]
        w0m = w0_ref[...]
        a1 = jnp.maximum(
            se_ref[...] @ w0m[:, :H].T + su_x @ w0m[:, H:].T + b0_ref[...],
            0.0)
        l = a1 @ w1_ref[...].T + b1_ref[...]
        e = jnp.exp(l)
        y_ref[...] = jnp.concatenate([su_x * e, e], axis=1)

    return pl.pallas_call(
        body,
        grid=(grid,),
        in_specs=[
            pl.BlockSpec((CE, H), lambda i: (i, 0)),
            pl.BlockSpec((CE, H), lambda i: (i, 0)),
            pl.BlockSpec((H, 2 * H), lambda i: (0, 0)),
            pl.BlockSpec((1, H), lambda i: (0, 0)),
            pl.BlockSpec((H, H), lambda i: (0, 0)),
            pl.BlockSpec((1, 1), lambda i: (0, 0)),
        ],
        out_specs=pl.BlockSpec((CE, YW), lambda i: (i, 0)),
        out_shape=jax.ShapeDtypeStruct((E_SOC, YW), jnp.float32),
    )(se, su, w0, b0, w1, b1)


def _sc_segsum(y_i, seg_i, y_u, seg_u, y_s, seg_s, zero_rows):
    outs = [jax.ShapeDtypeStruct((NC, B, YW), jnp.float32)] * 3

    @functools.partial(
        pl.kernel, out_type=outs, mesh=_mesh(),
        compiler_params=_SC_PARAMS,
        scratch_types=[pltpu.VMEM((GC, YW), jnp.float32),
                       pltpu.VMEM((GC,), jnp.int32),
                       pltpu.VMEM_SHARED((B, YW), jnp.float32),
                       pltpu.VMEM_SHARED((B, YW), jnp.float32),
                       pltpu.VMEM_SHARED((B, YW), jnp.float32)])
    def k(yi, si, yu, su_, ys, ss, z, out_i, out_u, out_s,
          y_v, seg_v, acc_i, acc_u, acc_s):
        c = lax.axis_index("c")
        s = lax.axis_index("s")
        w = s * NC + c

        @pl.when(s == 0)
        def _zero():
            pltpu.sync_copy(z, acc_i)
            pltpu.sync_copy(z, acc_u)
            pltpu.sync_copy(z, acc_s)

        plsc.subcore_barrier()

        def task(y, seg, acc, n):
            per = n // NW
            base = pl.multiple_of(w * per, 8)

            def body(i, carry):
                off = pl.multiple_of(base + i * GC, 8)
                pltpu.sync_copy(y.at[pl.ds(off, GC)], y_v)
                pltpu.sync_copy(seg.at[pl.ds(off, GC)], seg_v)
                pltpu.sync_copy(y_v, acc.at[seg_v], add=True)
                return carry

            lax.fori_loop(0, per // GC, body, 0)

        task(yi, si, acc_i, E_ITEM)
        task(yu, su_, acc_u, E_ITEM)
        task(ys, ss, acc_s, E_SOC)

        plsc.subcore_barrier()
        rows = B // NS
        sl = pl.ds(s * rows, rows)
        pltpu.sync_copy(acc_i.at[sl], out_i.at[c, sl])
        pltpu.sync_copy(acc_u.at[sl], out_u.at[c, sl])
        pltpu.sync_copy(acc_s.at[sl], out_s.at[c, sl])

    return k(y_i, seg_i, y_u, seg_u, y_s, seg_s, zero_rows)


CI = 2048


def _tc_final(agg_i, agg_u, agg_s, item_table,
              eq4_w, eq4_b, eq4i_w, eq4i_b, eq9_w, eq9_b, eq13_w, eq13_b,
              mlp_w0, mlp_b0, mlp_w1, mlp_b1, mlp_w2, mlp_b2):
    grid = pl.cdiv(item_table.shape[0], CI)

    def body(ai, au, asoc, it, e4w, e4b, e4iw, e4ib, e9w, e9b,
             e13w, e13b, m0w, m0b, m1w, m1b, m2w, m2b,
             m_ref, sc_ref, h_scr):
        pid = pl.program_id(0)

        @pl.when(pid == 0)
        def _head():
            def head(agg, wt, bt):
                a = agg[0] + agg[1]
                v = a[:, :H]
                sden = a[:, H:]
                return jnp.maximum((v / (sden + 1e-16)) @ wt[...].T + bt[...],
                                   0.0)

            hi = head(ai[...], e4w, e4b)
            zj = head(au[...], e4iw, e4ib)
            hs = head(asoc[...], e9w, e9b)
            e13 = e13w[...]
            h = jnp.maximum(hi @ e13[:, :H].T + hs @ e13[:, H:].T + e13b[...],
                            0.0)
            h_scr[...] = h
            m0 = m0w[...]
            mm = h @ m0[:, :H].T + zj @ m0[:, H:].T + m0b[...]
            mm = jnp.maximum(mm, 0.0) @ m1w[...].T + m1b[...]
            mm = jnp.maximum(mm, 0.0) @ m2w[...].T + m2b[...]
            m_ref[...] = mm[:, :1]

        sc_ref[...] = h_scr[...] @ it[...].T

    full = lambda i: (0, 0)
    full3 = lambda i: (0, 0, 0)
    return pl.pallas_call(
        body,
        grid=(grid,),
        in_specs=[
            pl.BlockSpec((NC, B, YW), full3),
            pl.BlockSpec((NC, B, YW), full3),
            pl.BlockSpec((NC, B, YW), full3),
            pl.BlockSpec((CI, H), lambda i: (i, 0)),
            pl.BlockSpec((H, H), full),
            pl.BlockSpec((1, H), full),
            pl.BlockSpec((H, H), full),
            pl.BlockSpec((1, H), full),
            pl.BlockSpec((H, H), full),
            pl.BlockSpec((1, H), full),
            pl.BlockSpec((H, 2 * H), full),
            pl.BlockSpec((1, H), full),
            pl.BlockSpec((H, 2 * H), full),
            pl.BlockSpec((1, H), full),
            pl.BlockSpec((H, H), full),
            pl.BlockSpec((1, H), full),
            pl.BlockSpec((H, H), full),
            pl.BlockSpec((1, 1), full),
        ],
        out_specs=[
            pl.BlockSpec((B, 1), lambda i: (0, 0)),
            pl.BlockSpec((B, CI), lambda i: (0, i)),
        ],
        out_shape=[
            jax.ShapeDtypeStruct((B, 1), jnp.float32),
            jax.ShapeDtypeStruct((B, item_table.shape[0]), jnp.float32),
        ],
        scratch_shapes=[pltpu.VMEM((B, H), jnp.float32)],
    )(agg_i, agg_u, agg_s, item_table,
      eq4_w, eq4_b, eq4i_w, eq4i_b, eq9_w, eq9_b, eq13_w, eq13_b,
      mlp_w0, mlp_b0, mlp_w1, mlp_b1, mlp_w2, mlp_b2)


def kernel(user, item, rating, item4user, social, social4user,
           batch_i_users, batch_i_ratings, batch_u_item, batch_target,
           user_table, item_table, rating_table,
           gv_w, gv_b, gu_w, gu_b,
           eq5_w0, eq5_b0, eq5_w1, eq5_b1,
           eq5i_w0, eq5i_b0, eq5i_w1, eq5i_b1,
           eq4_w, eq4_b, eq4i_w, eq4i_b,
           eq10_w0, eq10_b0, eq10_w1, eq10_b1,
           eq9_w, eq9_b, eq13_w, eq13_b,
           mlp_w0, mlp_b0, mlp_w1, mlp_b1, mlp_w2, mlp_b2):
    i32 = jnp.int32
    user = user.astype(i32)
    item = item.astype(i32)
    rating = rating.astype(i32)
    item4user = item4user.astype(i32)
    social = social.astype(i32)
    social4user = social4user.astype(i32)
    batch_i_users = batch_i_users.astype(i32)
    batch_i_ratings = batch_i_ratings.astype(i32)
    batch_u_item = batch_u_item.astype(i32)
    batch_target = batch_target.astype(i32)

    user_emb, tgt, item_emb, social_emb, item_u_emb = _sc_gather_stage1(
        user_table, item_table, user, batch_target, item, social,
        batch_i_users)
    ue_g, su, tgt_g = _sc_gather_stage2(
        user_emb, tgt, item4user, social4user, batch_u_item)

    r2 = lambda b: b.reshape(1, -1)
    rep = lambda w: jnp.broadcast_to(w, (H, H))
    rids3 = rating.reshape(E_ITEM // CE, 1, CE)
    brids3 = batch_i_ratings.reshape(E_ITEM // CE, 1, CE)
    y_i = _tc_edges_proj(item_emb, rids3, ue_g, rating_table,
                         gv_w, r2(gv_b), eq5_w0, r2(eq5_b0), rep(eq5_w1),
                         r2(eq5_b1), E_ITEM)
    y_u = _tc_edges_proj(item_u_emb, brids3, tgt_g, rating_table,
                         gu_w, r2(gu_b), eq5i_w0, r2(eq5i_b0), rep(eq5i_w1),
                         r2(eq5i_b1), E_ITEM)
    y_s = _tc_edges_soc(social_emb, su, eq10_w0, r2(eq10_b0), rep(eq10_w1),
                        r2(eq10_b1))

    zero_rows = jnp.zeros((B, YW), jnp.float32)
    agg_i, agg_u, agg_s = _sc_segsum(y_i, item4user, y_u, batch_u_item,
                                     y_s, social4user, zero_rows)

    m, scores = _tc_final(agg_i, agg_u, agg_s, item_table,
                          eq4_w, r2(eq4_b), eq4i_w, r2(eq4i_b),
                          eq9_w, r2(eq9_b), eq13_w, r2(eq13_b),
                          mlp_w0, r2(mlp_b0), mlp_w1, r2(mlp_b1),
                          rep(mlp_w2), mlp_b2.reshape(1, 1))
    return m, scores


# speedup vs baseline: 1.1030x; 1.1030x over previous
import functools

import jax
import jax.numpy as jnp
from jax import lax
from jax.experimental import pallas as pl
from jax.experimental.pallas import tpu as pltpu
from jax.experimental.pallas import tpu_sc as plsc

H = 64
B = 1024
E_ITEM = 51200
E_SOC = 20480
NC = 2
NS = 16
NW = NC * NS
GC = 80
YW = 2 * H


def _mesh():
    return plsc.VectorSubcoreMesh(core_axis_name="c", subcore_axis_name="s")


_SC_PARAMS = pltpu.CompilerParams(use_tc_tiling_on_sc=False)


def _worker_id():
    return lax.axis_index("s") * NC + lax.axis_index("c")


NBUF = 4


def _gather_task(table, idx, out, n, idx_all, bufs, sems, w):
    per = n // NW
    base = pl.multiple_of(w * per, 8)
    pltpu.sync_copy(idx.at[pl.ds(base, per)], idx_all.at[pl.ds(0, per)])
    if per >= GC * NBUF:
        def body(i, carry):
            c0 = i * NBUF
            hs = [
                pltpu.async_copy(
                    table.at[idx_all.at[pl.ds((c0 + j) * GC, GC)]],
                    bufs[j], sems[j])
                for j in range(NBUF)
            ]
            for j in range(NBUF):
                hs[j].wait()
                pltpu.sync_copy(bufs[j],
                                out.at[pl.ds(base + (c0 + j) * GC, GC)])
            return carry
        lax.fori_loop(0, per // (GC * NBUF), body, 0)
    else:
        sl = pl.ds(0, per)
        pltpu.async_copy(table.at[idx_all.at[sl]], bufs[0].at[sl],
                         sems[0]).wait()
        pltpu.sync_copy(bufs[0].at[sl], out.at[pl.ds(base, per)])


def _sc_gather_stage1(user_table, item_table, user, batch_target, item,
                      social, batch_i_users):
    outs = [
        jax.ShapeDtypeStruct((B, H), jnp.float32),
        jax.ShapeDtypeStruct((B, H), jnp.float32),
        jax.ShapeDtypeStruct((E_ITEM, H), jnp.float32),
        jax.ShapeDtypeStruct((E_SOC, H), jnp.float32),
        jax.ShapeDtypeStruct((E_ITEM, H), jnp.float32),
    ]

    @functools.partial(
        pl.kernel, out_type=outs, mesh=_mesh(),
        compiler_params=_SC_PARAMS,
        scratch_types=[pltpu.VMEM((E_ITEM // NW,), jnp.int32)]
                      + [pltpu.VMEM((GC, H), jnp.float32)] * NBUF
                      + [pltpu.SemaphoreType.DMA] * NBUF)
    def k(user_t, item_t, user_i, btgt_i, item_i, social_i, biu_i,
          user_emb, tgt, item_emb, social_emb, item_u_emb,
          idx_all, b0, b1, b2, b3, s0, s1, s2, s3):
        w = _worker_id()
        bufs, sems = [b0, b1, b2, b3], [s0, s1, s2, s3]
        _gather_task(user_t, user_i, user_emb, B, idx_all, bufs, sems, w)
        _gather_task(item_t, btgt_i, tgt, B, idx_all, bufs, sems, w)
        _gather_task(item_t, item_i, item_emb, E_ITEM, idx_all, bufs, sems, w)
        _gather_task(user_t, social_i, social_emb, E_SOC, idx_all, bufs, sems,
                     w)
        _gather_task(user_t, biu_i, item_u_emb, E_ITEM, idx_all, bufs, sems,
                     w)

    return k(user_table, item_table, user, batch_target, item, social,
             batch_i_users)


def _sc_gather_stage2(user_emb, tgt, item4user, social4user, batch_u_item):
    outs = [
        jax.ShapeDtypeStruct((E_ITEM, H), jnp.float32),
        jax.ShapeDtypeStruct((E_SOC, H), jnp.float32),
        jax.ShapeDtypeStruct((E_ITEM, H), jnp.float32),
    ]

    @functools.partial(
        pl.kernel, out_type=outs, mesh=_mesh(),
        compiler_params=_SC_PARAMS,
        scratch_types=[pltpu.VMEM((E_ITEM // NW,), jnp.int32)]
                      + [pltpu.VMEM((GC, H), jnp.float32)] * NBUF
                      + [pltpu.SemaphoreType.DMA] * NBUF)
    def k(ue_t, tgt_t, i4u_i, s4u_i, bui_i, ue_g, su, tgt_g,
          idx_all, b0, b1, b2, b3, s0, s1, s2, s3):
        w = _worker_id()
        bufs, sems = [b0, b1, b2, b3], [s0, s1, s2, s3]
        _gather_task(ue_t, i4u_i, ue_g, E_ITEM, idx_all, bufs, sems, w)
        _gather_task(ue_t, s4u_i, su, E_SOC, idx_all, bufs, sems, w)
        _gather_task(tgt_t, bui_i, tgt_g, E_ITEM, idx_all, bufs, sems, w)

    return k(user_emb, tgt, item4user, social4user, batch_u_item)


CE = 2048


def _tc_edges_proj(x1, rids3, xg, r_table, w_pre, b_pre, w0, b0, w1, b1, n):
    grid = n // CE

    def body(x1_ref, rid_ref, xg_ref, rt_ref, wpre_ref, bpre_ref,
             w0_ref, b0_ref, w1_ref, b1_ref, y_ref):
        ids = rid_ref[0, 0, :]
        oh = (ids[:, None] == lax.broadcasted_iota(jnp.int32, (CE, 16), 1)
              ).astype(jnp.float32)
        x2 = oh @ rt_ref[...]
        wpre = wpre_ref[...]
        xia = (x1_ref[...] @ wpre[:, :H].T + x2 @ wpre[:, H:].T
               + bpre_ref[...])
        w0m = w0_ref[...]
        a1 = jnp.maximum(
            xia @ w0m[:, :H].T + xg_ref[...] @ w0m[:, H:].T + b0_ref[...], 0.0)
        l = a1 @ w1_ref[...].T + b1_ref[...]
        e = jnp.exp(l)
        y_ref[...] = jnp.concatenate([xia * e, e], axis=1)

    rt16 = jnp.zeros((16, H), jnp.float32).at[:10].set(r_table)
    return pl.pallas_call(
        body,
        grid=(grid,),
        in_specs=[
            pl.BlockSpec((CE, H), lambda i: (i, 0)),
            pl.BlockSpec((1, 1, CE), lambda i: (i, 0, 0)),
            pl.BlockSpec((CE, H), lambda i: (i, 0)),
            pl.BlockSpec((16, H), lambda i: (0, 0)),
            pl.BlockSpec((H, 2 * H), lambda i: (0, 0)),
            pl.BlockSpec((1, H), lambda i: (0, 0)),
            pl.BlockSpec((H, 2 * H), lambda i: (0, 0)),
            pl.BlockSpec((1, H), lambda i: (0, 0)),
            pl.BlockSpec((H, H), lambda i: (0, 0)),
            pl.BlockSpec((1, 1), lambda i: (0, 0)),
        ],
        out_specs=pl.BlockSpec((CE, YW), lambda i: (i, 0)),
        out_shape=jax.ShapeDtypeStruct((n, YW), jnp.float32),
    )(x1, rids3, xg, rt16, w_pre, b_pre, w0, b0, w1, b1)


def _tc_edges_soc(se, su, w0, b0, w1, b1):
    grid = E_SOC // CE

    def body(se_ref, su_ref, w0_ref, b0_ref, w1_ref, b1_ref, y_ref):
        su_x = su_ref[...]
        w0m = w0_ref[...]
        a1 = jnp.maximum(
            se_ref[...] @ w0m[:, :H].T + su_x @ w0m[:, H:].T + b0_ref[...],
            0.0)
        l = a1 @ w1_ref[...].T + b1_ref[...]
        e = jnp.exp(l)
        y_ref[...] = jnp.concatenate([su_x * e, e], axis=1)

    return pl.pallas_call(
        body,
        grid=(grid,),
        in_specs=[
            pl.BlockSpec((CE, H), lambda i: (i, 0)),
            pl.BlockSpec((CE, H), lambda i: (i, 0)),
            pl.BlockSpec((H, 2 * H), lambda i: (0, 0)),
            pl.BlockSpec((1, H), lambda i: (0, 0)),
            pl.BlockSpec((H, H), lambda i: (0, 0)),
            pl.BlockSpec((1, 1), lambda i: (0, 0)),
        ],
        out_specs=pl.BlockSpec((CE, YW), lambda i: (i, 0)),
        out_shape=jax.ShapeDtypeStruct((E_SOC, YW), jnp.float32),
    )(se, su, w0, b0, w1, b1)


def _sc_segsum(y_i, seg_i, y_u, seg_u, y_s, seg_s, zero_rows):
    outs = [jax.ShapeDtypeStruct((NC, B, YW), jnp.float32)] * 3

    @functools.partial(
        pl.kernel, out_type=outs, mesh=_mesh(),
        compiler_params=_SC_PARAMS,
        scratch_types=[pltpu.VMEM((GC, YW), jnp.float32),
                       pltpu.VMEM((GC, YW), jnp.float32),
                       pltpu.VMEM((GC,), jnp.int32),
                       pltpu.VMEM((GC,), jnp.int32),
                       pltpu.SemaphoreType.DMA,
                       pltpu.SemaphoreType.DMA,
                       pltpu.VMEM_SHARED((B, YW), jnp.float32),
                       pltpu.VMEM_SHARED((B, YW), jnp.float32),
                       pltpu.VMEM_SHARED((B, YW), jnp.float32)])
    def k(yi, si, yu, su_, ys, ss, z, out_i, out_u, out_s,
          y_v, y_v1, seg_v, seg_v1, ysem0, ysem1, acc_i, acc_u, acc_s):
        c = lax.axis_index("c")
        s = lax.axis_index("s")
        w = s * NC + c

        @pl.when(s == 0)
        def _zero():
            pltpu.sync_copy(z, acc_i)
            pltpu.sync_copy(z, acc_u)
            pltpu.sync_copy(z, acc_s)

        plsc.subcore_barrier()

        def task(y, seg, acc, n):
            per = n // NW
            base = pl.multiple_of(w * per, 8)

            def body(i, carry):
                o0 = pl.multiple_of(base + (2 * i) * GC, 8)
                o1 = pl.multiple_of(base + (2 * i + 1) * GC, 8)
                h0 = pltpu.async_copy(y.at[pl.ds(o0, GC)], y_v, ysem0)
                h1 = pltpu.async_copy(y.at[pl.ds(o1, GC)], y_v1, ysem1)
                pltpu.sync_copy(seg.at[pl.ds(o0, GC)], seg_v)
                pltpu.sync_copy(seg.at[pl.ds(o1, GC)], seg_v1)
                h0.wait()
                pltpu.sync_copy(y_v, acc.at[seg_v], add=True)
                h1.wait()
                pltpu.sync_copy(y_v1, acc.at[seg_v1], add=True)
                return carry

            lax.fori_loop(0, per // (2 * GC), body, 0)

        task(yi, si, acc_i, E_ITEM)
        task(yu, su_, acc_u, E_ITEM)
        task(ys, ss, acc_s, E_SOC)

        plsc.subcore_barrier()
        rows = B // NS
        sl = pl.ds(s * rows, rows)
        pltpu.sync_copy(acc_i.at[sl], out_i.at[c, sl])
        pltpu.sync_copy(acc_u.at[sl], out_u.at[c, sl])
        pltpu.sync_copy(acc_s.at[sl], out_s.at[c, sl])

    return k(y_i, seg_i, y_u, seg_u, y_s, seg_s, zero_rows)


CI = 2048


def _tc_final(agg_i, agg_u, agg_s, item_table,
              eq4_w, eq4_b, eq4i_w, eq4i_b, eq9_w, eq9_b, eq13_w, eq13_b,
              mlp_w0, mlp_b0, mlp_w1, mlp_b1, mlp_w2, mlp_b2):
    grid = pl.cdiv(item_table.shape[0], CI)

    def body(ai, au, asoc, it, e4w, e4b, e4iw, e4ib, e9w, e9b,
             e13w, e13b, m0w, m0b, m1w, m1b, m2w, m2b,
             m_ref, sc_ref, h_scr):
        pid = pl.program_id(0)

        @pl.when(pid == 0)
        def _head():
            def head(agg, wt, bt):
                a = agg[0] + agg[1]
                v = a[:, :H]
                sden = a[:, H:]
                return jnp.maximum((v / (sden + 1e-16)) @ wt[...].T + bt[...],
                                   0.0)

            hi = head(ai[...], e4w, e4b)
            zj = head(au[...], e4iw, e4ib)
            hs = head(asoc[...], e9w, e9b)
            e13 = e13w[...]
            h = jnp.maximum(hi @ e13[:, :H].T + hs @ e13[:, H:].T + e13b[...],
                            0.0)
            h_scr[...] = h
            m0 = m0w[...]
            mm = h @ m0[:, :H].T + zj @ m0[:, H:].T + m0b[...]
            mm = jnp.maximum(mm, 0.0) @ m1w[...].T + m1b[...]
            mm = jnp.maximum(mm, 0.0) @ m2w[...].T + m2b[...]
            m_ref[...] = mm[:, :1]

        sc_ref[...] = h_scr[...] @ it[...].T

    full = lambda i: (0, 0)
    full3 = lambda i: (0, 0, 0)
    return pl.pallas_call(
        body,
        grid=(grid,),
        in_specs=[
            pl.BlockSpec((NC, B, YW), full3),
            pl.BlockSpec((NC, B, YW), full3),
            pl.BlockSpec((NC, B, YW), full3),
            pl.BlockSpec((CI, H), lambda i: (i, 0)),
            pl.BlockSpec((H, H), full),
            pl.BlockSpec((1, H), full),
            pl.BlockSpec((H, H), full),
            pl.BlockSpec((1, H), full),
            pl.BlockSpec((H, H), full),
            pl.BlockSpec((1, H), full),
            pl.BlockSpec((H, 2 * H), full),
            pl.BlockSpec((1, H), full),
            pl.BlockSpec((H, 2 * H), full),
            pl.BlockSpec((1, H), full),
            pl.BlockSpec((H, H), full),
            pl.BlockSpec((1, H), full),
            pl.BlockSpec((H, H), full),
            pl.BlockSpec((1, 1), full),
        ],
        out_specs=[
            pl.BlockSpec((B, 1), lambda i: (0, 0)),
            pl.BlockSpec((B, CI), lambda i: (0, i)),
        ],
        out_shape=[
            jax.ShapeDtypeStruct((B, 1), jnp.float32),
            jax.ShapeDtypeStruct((B, item_table.shape[0]), jnp.float32),
        ],
        scratch_shapes=[pltpu.VMEM((B, H), jnp.float32)],
    )(agg_i, agg_u, agg_s, item_table,
      eq4_w, eq4_b, eq4i_w, eq4i_b, eq9_w, eq9_b, eq13_w, eq13_b,
      mlp_w0, mlp_b0, mlp_w1, mlp_b1, mlp_w2, mlp_b2)


def kernel(user, item, rating, item4user, social, social4user,
           batch_i_users, batch_i_ratings, batch_u_item, batch_target,
           user_table, item_table, rating_table,
           gv_w, gv_b, gu_w, gu_b,
           eq5_w0, eq5_b0, eq5_w1, eq5_b1,
           eq5i_w0, eq5i_b0, eq5i_w1, eq5i_b1,
           eq4_w, eq4_b, eq4i_w, eq4i_b,
           eq10_w0, eq10_b0, eq10_w1, eq10_b1,
           eq9_w, eq9_b, eq13_w, eq13_b,
           mlp_w0, mlp_b0, mlp_w1, mlp_b1, mlp_w2, mlp_b2):
    i32 = jnp.int32
    user = user.astype(i32)
    item = item.astype(i32)
    rating = rating.astype(i32)
    item4user = item4user.astype(i32)
    social = social.astype(i32)
    social4user = social4user.astype(i32)
    batch_i_users = batch_i_users.astype(i32)
    batch_i_ratings = batch_i_ratings.astype(i32)
    batch_u_item = batch_u_item.astype(i32)
    batch_target = batch_target.astype(i32)

    user_emb, tgt, item_emb, social_emb, item_u_emb = _sc_gather_stage1(
        user_table, item_table, user, batch_target, item, social,
        batch_i_users)
    ue_g, su, tgt_g = _sc_gather_stage2(
        user_emb, tgt, item4user, social4user, batch_u_item)

    r2 = lambda b: b.reshape(1, -1)
    rep = lambda w: jnp.broadcast_to(w, (H, H))
    rids3 = rating.reshape(E_ITEM // CE, 1, CE)
    brids3 = batch_i_ratings.reshape(E_ITEM // CE, 1, CE)
    y_i = _tc_edges_proj(item_emb, rids3, ue_g, rating_table,
                         gv_w, r2(gv_b), eq5_w0, r2(eq5_b0), rep(eq5_w1),
                         r2(eq5_b1), E_ITEM)
    y_u = _tc_edges_proj(item_u_emb, brids3, tgt_g, rating_table,
                         gu_w, r2(gu_b), eq5i_w0, r2(eq5i_b0), rep(eq5i_w1),
                         r2(eq5i_b1), E_ITEM)
    y_s = _tc_edges_soc(social_emb, su, eq10_w0, r2(eq10_b0), rep(eq10_w1),
                        r2(eq10_b1))

    zero_rows = jnp.zeros((B, YW), jnp.float32)
    agg_i, agg_u, agg_s = _sc_segsum(y_i, item4user, y_u, batch_u_item,
                                     y_s, social4user, zero_rows)

    m, scores = _tc_final(agg_i, agg_u, agg_s, item_table,
                          eq4_w, r2(eq4_b), eq4i_w, r2(eq4i_b),
                          eq9_w, r2(eq9_b), eq13_w, r2(eq13_b),
                          mlp_w0, r2(mlp_b0), mlp_w1, r2(mlp_b1),
                          rep(mlp_w2), mlp_b2.reshape(1, 1))
    return m, scores


# speedup vs baseline: 1.4572x; 1.3212x over previous
import functools

import jax
import jax.numpy as jnp
from jax import lax
from jax.experimental import pallas as pl
from jax.experimental.pallas import tpu as pltpu
from jax.experimental.pallas import tpu_sc as plsc

H = 64
B = 1024
E_ITEM = 51200
E_SOC = 20480
NC = 2
NS = 16
NW = NC * NS
GC = 80
YW = 2 * H


def _mesh():
    return plsc.VectorSubcoreMesh(core_axis_name="c", subcore_axis_name="s")


_SC_PARAMS = pltpu.CompilerParams(use_tc_tiling_on_sc=False)


def _worker_id():
    return lax.axis_index("s") * NC + lax.axis_index("c")


NBUF = 4


def _gather_task(table, idx, out, n, idx_all, bufs, sems, w):
    D = table.shape[1]
    per = n // NW
    base = pl.multiple_of(w * per, 8)
    cols = pl.ds(0, D)
    pltpu.sync_copy(idx.at[pl.ds(base, per)], idx_all.at[pl.ds(0, per)])
    if per >= GC * NBUF:
        def body(i, carry):
            c0 = i * NBUF
            hs = [
                pltpu.async_copy(
                    table.at[idx_all.at[pl.ds((c0 + j) * GC, GC)]],
                    bufs[j], sems[j])
                for j in range(NBUF)
            ]
            for j in range(NBUF):
                hs[j].wait()
                pltpu.sync_copy(
                    bufs[j], out.at[pl.ds(base + (c0 + j) * GC, GC), cols])
            return carry
        lax.fori_loop(0, per // (GC * NBUF), body, 0)
    else:
        sl = pl.ds(0, per)
        pltpu.async_copy(table.at[idx_all.at[sl]], bufs[0].at[sl],
                         sems[0]).wait()
        pltpu.sync_copy(bufs[0].at[sl], out.at[pl.ds(base, per), cols])


def _sc_gather_stage1(user_table, item_table, user, batch_target, item,
                      social, batch_i_users):
    outs = [
        jax.ShapeDtypeStruct((B, YW), jnp.float32),
        jax.ShapeDtypeStruct((B, YW), jnp.float32),
        jax.ShapeDtypeStruct((E_ITEM, YW), jnp.float32),
        jax.ShapeDtypeStruct((E_SOC, YW), jnp.float32),
        jax.ShapeDtypeStruct((E_ITEM, YW), jnp.float32),
    ]

    @functools.partial(
        pl.kernel, out_type=outs, mesh=_mesh(),
        compiler_params=_SC_PARAMS,
        scratch_types=[pltpu.VMEM((E_ITEM // NW,), jnp.int32)]
                      + [pltpu.VMEM((GC, H), jnp.float32)] * NBUF
                      + [pltpu.SemaphoreType.DMA] * NBUF)
    def k(user_t, item_t, user_i, btgt_i, item_i, social_i, biu_i,
          user_emb, tgt, item_emb, social_emb, item_u_emb,
          idx_all, b0, b1, b2, b3, s0, s1, s2, s3):
        w = _worker_id()
        bufs, sems = [b0, b1, b2, b3], [s0, s1, s2, s3]
        _gather_task(user_t, user_i, user_emb, B, idx_all, bufs, sems, w)
        _gather_task(item_t, btgt_i, tgt, B, idx_all, bufs, sems, w)
        _gather_task(item_t, item_i, item_emb, E_ITEM, idx_all, bufs, sems, w)
        _gather_task(user_t, social_i, social_emb, E_SOC, idx_all, bufs, sems,
                     w)
        _gather_task(user_t, biu_i, item_u_emb, E_ITEM, idx_all, bufs, sems,
                     w)

    return k(user_table, item_table, user, batch_target, item, social,
             batch_i_users)


def _sc_gather_stage2(user_emb, tgt, item4user, social4user, batch_u_item):
    outs = [
        jax.ShapeDtypeStruct((E_ITEM, YW), jnp.float32),
        jax.ShapeDtypeStruct((E_SOC, YW), jnp.float32),
        jax.ShapeDtypeStruct((E_ITEM, YW), jnp.float32),
    ]

    @functools.partial(
        pl.kernel, out_type=outs, mesh=_mesh(),
        compiler_params=_SC_PARAMS,
        scratch_types=[pltpu.VMEM((E_ITEM // NW,), jnp.int32)]
                      + [pltpu.VMEM((GC, YW), jnp.float32)] * NBUF
                      + [pltpu.SemaphoreType.DMA] * NBUF)
    def k(ue_t, tgt_t, i4u_i, s4u_i, bui_i, ue_g, su, tgt_g,
          idx_all, b0, b1, b2, b3, s0, s1, s2, s3):
        w = _worker_id()
        bufs, sems = [b0, b1, b2, b3], [s0, s1, s2, s3]
        _gather_task(ue_t, i4u_i, ue_g, E_ITEM, idx_all, bufs, sems, w)
        _gather_task(ue_t, s4u_i, su, E_SOC, idx_all, bufs, sems, w)
        _gather_task(tgt_t, bui_i, tgt_g, E_ITEM, idx_all, bufs, sems, w)

    return k(user_emb, tgt, item4user, social4user, batch_u_item)


CE = 2048


def _tc_edges_proj(x1, rids3, xg, r_table, w_pre, b_pre, w0, b0, w1, b1, n):
    grid = n // CE

    def body(x1_ref, rid_ref, xg_ref, rt_ref, wpre_ref, bpre_ref,
             w0_ref, b0_ref, w1_ref, b1_ref, y_ref):
        ids = rid_ref[0, 0, :]
        oh = (ids[:, None] == lax.broadcasted_iota(jnp.int32, (CE, 16), 1)
              ).astype(jnp.float32)
        x2 = oh @ rt_ref[...]
        wpre = wpre_ref[...]
        xia = (x1_ref[:, :H] @ wpre[:, :H].T + x2 @ wpre[:, H:].T
               + bpre_ref[...])
        w0m = w0_ref[...]
        a1 = jnp.maximum(
            xia @ w0m[:, :H].T + xg_ref[:, :H] @ w0m[:, H:].T + b0_ref[...],
            0.0)
        l = a1 @ w1_ref[...].T + b1_ref[...]
        e = jnp.exp(l)
        y_ref[...] = jnp.concatenate([xia * e, e], axis=1)

    rt16 = jnp.zeros((16, H), jnp.float32).at[:10].set(r_table)
    return pl.pallas_call(
        body,
        grid=(grid,),
        in_specs=[
            pl.BlockSpec((CE, YW), lambda i: (i, 0)),
            pl.BlockSpec((1, 1, CE), lambda i: (i, 0, 0)),
            pl.BlockSpec((CE, YW), lambda i: (i, 0)),
            pl.BlockSpec((16, H), lambda i: (0, 0)),
            pl.BlockSpec((H, 2 * H), lambda i: (0, 0)),
            pl.BlockSpec((1, H), lambda i: (0, 0)),
            pl.BlockSpec((H, 2 * H), lambda i: (0, 0)),
            pl.BlockSpec((1, H), lambda i: (0, 0)),
            pl.BlockSpec((H, H), lambda i: (0, 0)),
            pl.BlockSpec((1, 1), lambda i: (0, 0)),
        ],
        out_specs=pl.BlockSpec((CE, YW), lambda i: (i, 0)),
        out_shape=jax.ShapeDtypeStruct((n, YW), jnp.float32),
    )(x1, rids3, xg, rt16, w_pre, b_pre, w0, b0, w1, b1)


def _tc_edges_soc(se, su, w0, b0, w1, b1):
    grid = E_SOC // CE

    def body(se_ref, su_ref, w0_ref, b0_ref, w1_ref, b1_ref, y_ref):
        su_x = su_ref[:, :H]
        w0m = w0_ref[...]
        a1 = jnp.maximum(
            se_ref[:, :H] @ w0m[:, :H].T + su_x @ w0m[:, H:].T + b0_ref[...],
            0.0)
        l = a1 @ w1_ref[...].T + b1_ref[...]
        e = jnp.exp(l)
        y_ref[...] = jnp.concatenate([su_x * e, e], axis=1)

    return pl.pallas_call(
        body,
        grid=(grid,),
        in_specs=[
            pl.BlockSpec((CE, YW), lambda i: (i, 0)),
            pl.BlockSpec((CE, YW), lambda i: (i, 0)),
            pl.BlockSpec((H, 2 * H), lambda i: (0, 0)),
            pl.BlockSpec((1, H), lambda i: (0, 0)),
            pl.BlockSpec((H, H), lambda i: (0, 0)),
            pl.BlockSpec((1, 1), lambda i: (0, 0)),
        ],
        out_specs=pl.BlockSpec((CE, YW), lambda i: (i, 0)),
        out_shape=jax.ShapeDtypeStruct((E_SOC, YW), jnp.float32),
    )(se, su, w0, b0, w1, b1)


def _sc_segsum(y_i, seg_i, y_u, seg_u, y_s, seg_s, zero_rows):
    outs = [jax.ShapeDtypeStruct((NC, B, YW), jnp.float32)] * 3

    @functools.partial(
        pl.kernel, out_type=outs, mesh=_mesh(),
        compiler_params=_SC_PARAMS,
        scratch_types=[pltpu.VMEM((GC, YW), jnp.float32),
                       pltpu.VMEM((GC, YW), jnp.float32),
                       pltpu.VMEM((GC,), jnp.int32),
                       pltpu.VMEM((GC,), jnp.int32),
                       pltpu.SemaphoreType.DMA,
                       pltpu.SemaphoreType.DMA,
                       pltpu.VMEM_SHARED((B, YW), jnp.float32),
                       pltpu.VMEM_SHARED((B, YW), jnp.float32),
                       pltpu.VMEM_SHARED((B, YW), jnp.float32)])
    def k(yi, si, yu, su_, ys, ss, z, out_i, out_u, out_s,
          y_v, y_v1, seg_v, seg_v1, ysem0, ysem1, acc_i, acc_u, acc_s):
        c = lax.axis_index("c")
        s = lax.axis_index("s")
        w = s * NC + c

        @pl.when(s == 0)
        def _zero():
            pltpu.sync_copy(z, acc_i)
            pltpu.sync_copy(z, acc_u)
            pltpu.sync_copy(z, acc_s)

        plsc.subcore_barrier()

        def task(y, seg, acc, n):
            per = n // NW
            base = pl.multiple_of(w * per, 8)

            def body(i, carry):
                o0 = pl.multiple_of(base + (2 * i) * GC, 8)
                o1 = pl.multiple_of(base + (2 * i + 1) * GC, 8)
                h0 = pltpu.async_copy(y.at[pl.ds(o0, GC)], y_v, ysem0)
                h1 = pltpu.async_copy(y.at[pl.ds(o1, GC)], y_v1, ysem1)
                pltpu.sync_copy(seg.at[pl.ds(o0, GC)], seg_v)
                pltpu.sync_copy(seg.at[pl.ds(o1, GC)], seg_v1)
                h0.wait()
                pltpu.sync_copy(y_v, acc.at[seg_v], add=True)
                h1.wait()
                pltpu.sync_copy(y_v1, acc.at[seg_v1], add=True)
                return carry

            lax.fori_loop(0, per // (2 * GC), body, 0)

        task(yi, si, acc_i, E_ITEM)
        task(yu, su_, acc_u, E_ITEM)
        task(ys, ss, acc_s, E_SOC)

        plsc.subcore_barrier()
        rows = B // NS
        sl = pl.ds(s * rows, rows)
        pltpu.sync_copy(acc_i.at[sl], out_i.at[c, sl])
        pltpu.sync_copy(acc_u.at[sl], out_u.at[c, sl])
        pltpu.sync_copy(acc_s.at[sl], out_s.at[c, sl])

    return k(y_i, seg_i, y_u, seg_u, y_s, seg_s, zero_rows)


CI = 2048


def _tc_final(agg_i, agg_u, agg_s, item_table,
              eq4_w, eq4_b, eq4i_w, eq4i_b, eq9_w, eq9_b, eq13_w, eq13_b,
              mlp_w0, mlp_b0, mlp_w1, mlp_b1, mlp_w2, mlp_b2):
    grid = pl.cdiv(item_table.shape[0], CI)

    def body(ai, au, asoc, it, e4w, e4b, e4iw, e4ib, e9w, e9b,
             e13w, e13b, m0w, m0b, m1w, m1b, m2w, m2b,
             m_ref, sc_ref, h_scr):
        pid = pl.program_id(0)

        @pl.when(pid == 0)
        def _head():
            def head(agg, wt, bt):
                a = agg[0] + agg[1]
                v = a[:, :H]
                sden = a[:, H:]
                return jnp.maximum((v / (sden + 1e-16)) @ wt[...].T + bt[...],
                                   0.0)

            hi = head(ai[...], e4w, e4b)
            zj = head(au[...], e4iw, e4ib)
            hs = head(asoc[...], e9w, e9b)
            e13 = e13w[...]
            h = jnp.maximum(hi @ e13[:, :H].T + hs @ e13[:, H:].T + e13b[...],
                            0.0)
            h_scr[...] = h
            m0 = m0w[...]
            mm = h @ m0[:, :H].T + zj @ m0[:, H:].T + m0b[...]
            mm = jnp.maximum(mm, 0.0) @ m1w[...].T + m1b[...]
            mm = jnp.maximum(mm, 0.0) @ m2w[...].T + m2b[...]
            m_ref[...] = mm[:, :1]

        sc_ref[...] = lax.dot_general(it[...], h_scr[...],
                                      (((1,), (1,)), ((), ())))

    full = lambda i: (0, 0)
    full3 = lambda i: (0, 0, 0)
    return pl.pallas_call(
        body,
        grid=(grid,),
        in_specs=[
            pl.BlockSpec((NC, B, YW), full3),
            pl.BlockSpec((NC, B, YW), full3),
            pl.BlockSpec((NC, B, YW), full3),
            pl.BlockSpec((CI, H), lambda i: (i, 0)),
            pl.BlockSpec((H, H), full),
            pl.BlockSpec((1, H), full),
            pl.BlockSpec((H, H), full),
            pl.BlockSpec((1, H), full),
            pl.BlockSpec((H, H), full),
            pl.BlockSpec((1, H), full),
            pl.BlockSpec((H, 2 * H), full),
            pl.BlockSpec((1, H), full),
            pl.BlockSpec((H, 2 * H), full),
            pl.BlockSpec((1, H), full),
            pl.BlockSpec((H, H), full),
            pl.BlockSpec((1, H), full),
            pl.BlockSpec((H, H), full),
            pl.BlockSpec((1, 1), full),
        ],
        out_specs=[
            pl.BlockSpec((B, 1), lambda i: (0, 0)),
            pl.BlockSpec((CI, B), lambda i: (i, 0)),
        ],
        out_shape=[
            jax.ShapeDtypeStruct((B, 1), jnp.float32),
            jax.ShapeDtypeStruct((item_table.shape[0], B), jnp.float32),
        ],
        scratch_shapes=[pltpu.VMEM((B, H), jnp.float32)],
    )(agg_i, agg_u, agg_s, item_table,
      eq4_w, eq4_b, eq4i_w, eq4i_b, eq9_w, eq9_b, eq13_w, eq13_b,
      mlp_w0, mlp_b0, mlp_w1, mlp_b1, mlp_w2, mlp_b2)


def kernel(user, item, rating, item4user, social, social4user,
           batch_i_users, batch_i_ratings, batch_u_item, batch_target,
           user_table, item_table, rating_table,
           gv_w, gv_b, gu_w, gu_b,
           eq5_w0, eq5_b0, eq5_w1, eq5_b1,
           eq5i_w0, eq5i_b0, eq5i_w1, eq5i_b1,
           eq4_w, eq4_b, eq4i_w, eq4i_b,
           eq10_w0, eq10_b0, eq10_w1, eq10_b1,
           eq9_w, eq9_b, eq13_w, eq13_b,
           mlp_w0, mlp_b0, mlp_w1, mlp_b1, mlp_w2, mlp_b2):
    i32 = jnp.int32
    user = user.astype(i32)
    item = item.astype(i32)
    rating = rating.astype(i32)
    item4user = item4user.astype(i32)
    social = social.astype(i32)
    social4user = social4user.astype(i32)
    batch_i_users = batch_i_users.astype(i32)
    batch_i_ratings = batch_i_ratings.astype(i32)
    batch_u_item = batch_u_item.astype(i32)
    batch_target = batch_target.astype(i32)

    user_emb, tgt, item_emb, social_emb, item_u_emb = _sc_gather_stage1(
        user_table, item_table, user, batch_target, item, social,
        batch_i_users)
    ue_g, su, tgt_g = _sc_gather_stage2(
        user_emb, tgt, item4user, social4user, batch_u_item)

    r2 = lambda b: b.reshape(1, -1)
    rep = lambda w: jnp.broadcast_to(w, (H, H))
    rids3 = rating.reshape(E_ITEM // CE, 1, CE)
    brids3 = batch_i_ratings.reshape(E_ITEM // CE, 1, CE)
    y_i = _tc_edges_proj(item_emb, rids3, ue_g, rating_table,
                         gv_w, r2(gv_b), eq5_w0, r2(eq5_b0), rep(eq5_w1),
                         r2(eq5_b1), E_ITEM)
    y_u = _tc_edges_proj(item_u_emb, brids3, tgt_g, rating_table,
                         gu_w, r2(gu_b), eq5i_w0, r2(eq5i_b0), rep(eq5i_w1),
                         r2(eq5i_b1), E_ITEM)
    y_s = _tc_edges_soc(social_emb, su, eq10_w0, r2(eq10_b0), rep(eq10_w1),
                        r2(eq10_b1))

    zero_rows = jnp.zeros((B, YW), jnp.float32)
    agg_i, agg_u, agg_s = _sc_segsum(y_i, item4user, y_u, batch_u_item,
                                     y_s, social4user, zero_rows)

    m, scores_t = _tc_final(agg_i, agg_u, agg_s, item_table,
                            eq4_w, r2(eq4_b), eq4i_w, r2(eq4i_b),
                            eq9_w, r2(eq9_b), eq13_w, r2(eq13_b),
                            mlp_w0, r2(mlp_b0), mlp_w1, r2(mlp_b1),
                            rep(mlp_w2), mlp_b2.reshape(1, 1))
    return m, scores_t.T


# speedup vs baseline: 1.4589x; 1.0012x over previous
import functools

import jax
import jax.numpy as jnp
from jax import lax
from jax.experimental import pallas as pl
from jax.experimental.pallas import tpu as pltpu
from jax.experimental.pallas import tpu_sc as plsc

H = 64
B = 1024
E_ITEM = 51200
E_SOC = 20480
NC = 2
NS = 16
NW = NC * NS
GC = 80
YW = 2 * H


def _mesh():
    return plsc.VectorSubcoreMesh(core_axis_name="c", subcore_axis_name="s")


_SC_PARAMS = pltpu.CompilerParams(use_tc_tiling_on_sc=False)


def _worker_id():
    return lax.axis_index("s") * NC + lax.axis_index("c")


NBUF = 4


def _gather_task(table, idx, out, n, idx_all, bufs, sems, w):
    D = table.shape[1]
    per = n // NW
    base = pl.multiple_of(w * per, 8)
    cols = pl.ds(0, D)
    pltpu.sync_copy(idx.at[pl.ds(base, per)], idx_all.at[pl.ds(0, per)])
    if per >= GC * NBUF:
        def body(i, carry):
            c0 = i * NBUF
            hs = [
                pltpu.async_copy(
                    table.at[idx_all.at[pl.ds((c0 + j) * GC, GC)]],
                    bufs[j], sems[j])
                for j in range(NBUF)
            ]
            for j in range(NBUF):
                hs[j].wait()
                pltpu.sync_copy(
                    bufs[j], out.at[pl.ds(base + (c0 + j) * GC, GC), cols])
            return carry
        lax.fori_loop(0, per // (GC * NBUF), body, 0)
    else:
        sl = pl.ds(0, per)
        pltpu.async_copy(table.at[idx_all.at[sl]], bufs[0].at[sl],
                         sems[0]).wait()
        pltpu.sync_copy(bufs[0].at[sl], out.at[pl.ds(base, per), cols])


def _sc_gather_stage1(user_table, item_table, user, batch_target, item,
                      social, batch_i_users):
    outs = [
        jax.ShapeDtypeStruct((B, YW), jnp.float32),
        jax.ShapeDtypeStruct((B, YW), jnp.float32),
        jax.ShapeDtypeStruct((E_ITEM, YW), jnp.float32),
        jax.ShapeDtypeStruct((E_SOC, YW), jnp.float32),
        jax.ShapeDtypeStruct((E_ITEM, YW), jnp.float32),
    ]

    @functools.partial(
        pl.kernel, out_type=outs, mesh=_mesh(),
        compiler_params=_SC_PARAMS,
        scratch_types=[pltpu.VMEM((E_ITEM // NW,), jnp.int32)]
                      + [pltpu.VMEM((GC, H), jnp.float32)] * NBUF
                      + [pltpu.SemaphoreType.DMA] * NBUF)
    def k(user_t, item_t, user_i, btgt_i, item_i, social_i, biu_i,
          user_emb, tgt, item_emb, social_emb, item_u_emb,
          idx_all, b0, b1, b2, b3, s0, s1, s2, s3):
        w = _worker_id()
        bufs, sems = [b0, b1, b2, b3], [s0, s1, s2, s3]
        _gather_task(user_t, user_i, user_emb, B, idx_all, bufs, sems, w)
        _gather_task(item_t, btgt_i, tgt, B, idx_all, bufs, sems, w)
        _gather_task(item_t, item_i, item_emb, E_ITEM, idx_all, bufs, sems, w)
        _gather_task(user_t, social_i, social_emb, E_SOC, idx_all, bufs, sems,
                     w)
        _gather_task(user_t, biu_i, item_u_emb, E_ITEM, idx_all, bufs, sems,
                     w)

    return k(user_table, item_table, user, batch_target, item, social,
             batch_i_users)


def _sc_gather_stage2(user_emb, tgt, item4user, social4user, batch_u_item):
    outs = [
        jax.ShapeDtypeStruct((E_ITEM, YW), jnp.float32),
        jax.ShapeDtypeStruct((E_SOC, YW), jnp.float32),
        jax.ShapeDtypeStruct((E_ITEM, YW), jnp.float32),
    ]

    @functools.partial(
        pl.kernel, out_type=outs, mesh=_mesh(),
        compiler_params=_SC_PARAMS,
        scratch_types=[pltpu.VMEM((E_ITEM // NW,), jnp.int32)]
                      + [pltpu.VMEM((GC, YW), jnp.float32)] * NBUF
                      + [pltpu.SemaphoreType.DMA] * NBUF)
    def k(ue_t, tgt_t, i4u_i, s4u_i, bui_i, ue_g, su, tgt_g,
          idx_all, b0, b1, b2, b3, s0, s1, s2, s3):
        w = _worker_id()
        bufs, sems = [b0, b1, b2, b3], [s0, s1, s2, s3]
        _gather_task(ue_t, i4u_i, ue_g, E_ITEM, idx_all, bufs, sems, w)
        _gather_task(ue_t, s4u_i, su, E_SOC, idx_all, bufs, sems, w)
        _gather_task(tgt_t, bui_i, tgt_g, E_ITEM, idx_all, bufs, sems, w)

    return k(user_emb, tgt, item4user, social4user, batch_u_item)


CE = 2048


def _tc_edges_proj(x1, rids3, xg, r_table, w_pre, b_pre, w0, b0, w1, b1, n):
    grid = n // CE

    def body(x1_ref, rid_ref, xg_ref, rt_ref, wpre_ref, bpre_ref,
             w0_ref, b0_ref, w1_ref, b1_ref, y_ref):
        ids = rid_ref[0, 0, :]
        oh = (ids[:, None] == lax.broadcasted_iota(jnp.int32, (CE, 16), 1)
              ).astype(jnp.float32)
        x2 = oh @ rt_ref[...]
        wpre = wpre_ref[...]
        xia = (x1_ref[:, :H] @ wpre[:, :H].T + x2 @ wpre[:, H:].T
               + bpre_ref[...])
        w0m = w0_ref[...]
        a1 = jnp.maximum(
            xia @ w0m[:, :H].T + xg_ref[:, :H] @ w0m[:, H:].T + b0_ref[...],
            0.0)
        l = a1 @ w1_ref[...].T + b1_ref[...]
        e = jnp.exp(l)
        y_ref[...] = jnp.concatenate([xia * e, e], axis=1)

    rt16 = jnp.zeros((16, H), jnp.float32).at[:10].set(r_table)
    return pl.pallas_call(
        body,
        grid=(grid,),
        in_specs=[
            pl.BlockSpec((CE, YW), lambda i: (i, 0)),
            pl.BlockSpec((1, 1, CE), lambda i: (i, 0, 0)),
            pl.BlockSpec((CE, YW), lambda i: (i, 0)),
            pl.BlockSpec((16, H), lambda i: (0, 0)),
            pl.BlockSpec((H, 2 * H), lambda i: (0, 0)),
            pl.BlockSpec((1, H), lambda i: (0, 0)),
            pl.BlockSpec((H, 2 * H), lambda i: (0, 0)),
            pl.BlockSpec((1, H), lambda i: (0, 0)),
            pl.BlockSpec((H, H), lambda i: (0, 0)),
            pl.BlockSpec((1, 1), lambda i: (0, 0)),
        ],
        out_specs=pl.BlockSpec((CE, YW), lambda i: (i, 0)),
        out_shape=jax.ShapeDtypeStruct((n, YW), jnp.float32),
    )(x1, rids3, xg, rt16, w_pre, b_pre, w0, b0, w1, b1)


def _tc_edges_soc(se, su, w0, b0, w1, b1):
    grid = E_SOC // CE

    def body(se_ref, su_ref, w0_ref, b0_ref, w1_ref, b1_ref, y_ref):
        su_x = su_ref[:, :H]
        w0m = w0_ref[...]
        a1 = jnp.maximum(
            se_ref[:, :H] @ w0m[:, :H].T + su_x @ w0m[:, H:].T + b0_ref[...],
            0.0)
        l = a1 @ w1_ref[...].T + b1_ref[...]
        e = jnp.exp(l)
        y_ref[...] = jnp.concatenate([su_x * e, e], axis=1)

    return pl.pallas_call(
        body,
        grid=(grid,),
        in_specs=[
            pl.BlockSpec((CE, YW), lambda i: (i, 0)),
            pl.BlockSpec((CE, YW), lambda i: (i, 0)),
            pl.BlockSpec((H, 2 * H), lambda i: (0, 0)),
            pl.BlockSpec((1, H), lambda i: (0, 0)),
            pl.BlockSpec((H, H), lambda i: (0, 0)),
            pl.BlockSpec((1, 1), lambda i: (0, 0)),
        ],
        out_specs=pl.BlockSpec((CE, YW), lambda i: (i, 0)),
        out_shape=jax.ShapeDtypeStruct((E_SOC, YW), jnp.float32),
    )(se, su, w0, b0, w1, b1)


def _sc_segsum(y_i, seg_i, y_u, seg_u, y_s, seg_s, zero_rows):
    outs = [jax.ShapeDtypeStruct((NC, B, YW), jnp.float32)] * 3

    @functools.partial(
        pl.kernel, out_type=outs, mesh=_mesh(),
        compiler_params=_SC_PARAMS,
        scratch_types=[pltpu.VMEM((GC, YW), jnp.float32),
                       pltpu.VMEM((GC, YW), jnp.float32),
                       pltpu.VMEM((GC,), jnp.int32),
                       pltpu.VMEM((GC,), jnp.int32),
                       pltpu.SemaphoreType.DMA,
                       pltpu.SemaphoreType.DMA,
                       pltpu.VMEM_SHARED((B, YW), jnp.float32),
                       pltpu.VMEM_SHARED((B, YW), jnp.float32),
                       pltpu.VMEM_SHARED((B, YW), jnp.float32)])
    def k(yi, si, yu, su_, ys, ss, z, out_i, out_u, out_s,
          y_v, y_v1, seg_v, seg_v1, ysem0, ysem1, acc_i, acc_u, acc_s):
        c = lax.axis_index("c")
        s = lax.axis_index("s")
        w = s * NC + c

        @pl.when(s == 0)
        def _zero():
            pltpu.sync_copy(z, acc_i)
            pltpu.sync_copy(z, acc_u)
            pltpu.sync_copy(z, acc_s)

        plsc.subcore_barrier()

        def task(y, seg, acc, n):
            per = n // NW
            base = pl.multiple_of(w * per, 8)

            def body(i, carry):
                o0 = pl.multiple_of(base + (2 * i) * GC, 8)
                o1 = pl.multiple_of(base + (2 * i + 1) * GC, 8)
                h0 = pltpu.async_copy(y.at[pl.ds(o0, GC)], y_v, ysem0)
                h1 = pltpu.async_copy(y.at[pl.ds(o1, GC)], y_v1, ysem1)
                pltpu.sync_copy(seg.at[pl.ds(o0, GC)], seg_v)
                pltpu.sync_copy(seg.at[pl.ds(o1, GC)], seg_v1)
                h0.wait()
                pltpu.sync_copy(y_v, acc.at[seg_v], add=True)
                h1.wait()
                pltpu.sync_copy(y_v1, acc.at[seg_v1], add=True)
                return carry

            lax.fori_loop(0, per // (2 * GC), body, 0)

        task(yi, si, acc_i, E_ITEM)
        task(yu, su_, acc_u, E_ITEM)
        task(ys, ss, acc_s, E_SOC)

        plsc.subcore_barrier()
        rows = B // NS
        sl = pl.ds(s * rows, rows)
        pltpu.sync_copy(acc_i.at[sl], out_i.at[c, sl])
        pltpu.sync_copy(acc_u.at[sl], out_u.at[c, sl])
        pltpu.sync_copy(acc_s.at[sl], out_s.at[c, sl])

    return k(y_i, seg_i, y_u, seg_u, y_s, seg_s, zero_rows)


CI = 2048


def _tc_final(agg_i, agg_u, agg_s, item_table,
              eq4_w, eq4_b, eq4i_w, eq4i_b, eq9_w, eq9_b, eq13_w, eq13_b,
              mlp_w0, mlp_b0, mlp_w1, mlp_b1, mlp_w2, mlp_b2):
    grid = pl.cdiv(item_table.shape[0], CI)

    def body(ai, au, asoc, it, e4w, e4b, e4iw, e4ib, e9w, e9b,
             e13w, e13b, m0w, m0b, m1w, m1b, m2w, m2b,
             m_ref, sc_ref, h_scr):
        pid = pl.program_id(0)

        @pl.when(pid == 0)
        def _head():
            def head(agg, wt, bt):
                a = agg[0] + agg[1]
                v = a[:, :H]
                sden = a[:, H:]
                return jnp.maximum((v / (sden + 1e-16)) @ wt[...].T + bt[...],
                                   0.0)

            hi = head(ai[...], e4w, e4b)
            zj = head(au[...], e4iw, e4ib)
            hs = head(asoc[...], e9w, e9b)
            e13 = e13w[...]
            h = jnp.maximum(hi @ e13[:, :H].T + hs @ e13[:, H:].T + e13b[...],
                            0.0)
            h_scr[...] = h
            m0 = m0w[...]
            mm = h @ m0[:, :H].T + zj @ m0[:, H:].T + m0b[...]
            mm = jnp.maximum(mm, 0.0) @ m1w[...].T + m1b[...]
            mm = jnp.maximum(mm, 0.0) @ m2w[...].T + m2b[...]
            m_ref[...] = mm[:, :1]

        sc_ref[...] = lax.dot_general(it[...], h_scr[...],
                                      (((1,), (1,)), ((), ())))

    full = lambda i: (0, 0)
    full3 = lambda i: (0, 0, 0)
    return pl.pallas_call(
        body,
        grid=(grid,),
        in_specs=[
            pl.BlockSpec((NC, B, YW), full3),
            pl.BlockSpec((NC, B, YW), full3),
            pl.BlockSpec((NC, B, YW), full3),
            pl.BlockSpec((CI, H), lambda i: (i, 0)),
            pl.BlockSpec((H, H), full),
            pl.BlockSpec((1, H), full),
            pl.BlockSpec((H, H), full),
            pl.BlockSpec((1, H), full),
            pl.BlockSpec((H, H), full),
            pl.BlockSpec((1, H), full),
            pl.BlockSpec((H, 2 * H), full),
            pl.BlockSpec((1, H), full),
            pl.BlockSpec((H, 2 * H), full),
            pl.BlockSpec((1, H), full),
            pl.BlockSpec((H, H), full),
            pl.BlockSpec((1, H), full),
            pl.BlockSpec((H, H), full),
            pl.BlockSpec((1, 1), full),
        ],
        out_specs=[
            pl.BlockSpec((B, 1), lambda i: (0, 0)),
            pl.BlockSpec((CI, B), lambda i: (i, 0)),
        ],
        out_shape=[
            jax.ShapeDtypeStruct((B, 1), jnp.float32),
            jax.ShapeDtypeStruct((item_table.shape[0], B), jnp.float32),
        ],
        scratch_shapes=[pltpu.VMEM((B, H), jnp.float32)],
    )(agg_i, agg_u, agg_s, item_table,
      eq4_w, eq4_b, eq4i_w, eq4i_b, eq9_w, eq9_b, eq13_w, eq13_b,
      mlp_w0, mlp_b0, mlp_w1, mlp_b1, mlp_w2, mlp_b2)


def kernel(user, item, rating, item4user, social, social4user,
           batch_i_users, batch_i_ratings, batch_u_item, batch_target,
           user_table, item_table, rating_table,
           gv_w, gv_b, gu_w, gu_b,
           eq5_w0, eq5_b0, eq5_w1, eq5_b1,
           eq5i_w0, eq5i_b0, eq5i_w1, eq5i_b1,
           eq4_w, eq4_b, eq4i_w, eq4i_b,
           eq10_w0, eq10_b0, eq10_w1, eq10_b1,
           eq9_w, eq9_b, eq13_w, eq13_b,
           mlp_w0, mlp_b0, mlp_w1, mlp_b1, mlp_w2, mlp_b2):
    i32 = jnp.int32
    user = user.astype(i32)
    item = item.astype(i32)
    rating = rating.astype(i32)
    item4user = item4user.astype(i32)
    social = social.astype(i32)
    social4user = social4user.astype(i32)
    batch_i_users = batch_i_users.astype(i32)
    batch_i_ratings = batch_i_ratings.astype(i32)
    batch_u_item = batch_u_item.astype(i32)
    batch_target = batch_target.astype(i32)

    ut = lax.optimization_barrier(user_table.reshape(-1)).reshape(
        user_table.shape)
    user_emb, tgt, item_emb, social_emb, item_u_emb = _sc_gather_stage1(
        ut, item_table, user, batch_target, item, social,
        batch_i_users)
    ue_g, su, tgt_g = _sc_gather_stage2(
        user_emb, tgt, item4user, social4user, batch_u_item)

    r2 = lambda b: b.reshape(1, -1)
    rep = lambda w: jnp.broadcast_to(w, (H, H))
    rids3 = rating.reshape(E_ITEM // CE, 1, CE)
    brids3 = batch_i_ratings.reshape(E_ITEM // CE, 1, CE)
    y_i = _tc_edges_proj(item_emb, rids3, ue_g, rating_table,
                         gv_w, r2(gv_b), eq5_w0, r2(eq5_b0), rep(eq5_w1),
                         r2(eq5_b1), E_ITEM)
    y_u = _tc_edges_proj(item_u_emb, brids3, tgt_g, rating_table,
                         gu_w, r2(gu_b), eq5i_w0, r2(eq5i_b0), rep(eq5i_w1),
                         r2(eq5i_b1), E_ITEM)
    y_s = _tc_edges_soc(social_emb, su, eq10_w0, r2(eq10_b0), rep(eq10_w1),
                        r2(eq10_b1))

    zero_rows = jnp.zeros((B, YW), jnp.float32)
    agg_i, agg_u, agg_s = _sc_segsum(y_i, item4user, y_u, batch_u_item,
                                     y_s, social4user, zero_rows)

    m, scores_t = _tc_final(agg_i, agg_u, agg_s, item_table,
                            eq4_w, r2(eq4_b), eq4i_w, r2(eq4i_b),
                            eq9_w, r2(eq9_b), eq13_w, r2(eq13_b),
                            mlp_w0, r2(mlp_b0), mlp_w1, r2(mlp_b1),
                            rep(mlp_w2), mlp_b2.reshape(1, 1))
    return m, scores_t.T


# speedup vs baseline: 1.5033x; 1.0305x over previous
import functools

import jax
import jax.numpy as jnp
from jax import lax
from jax.experimental import pallas as pl
from jax.experimental.pallas import tpu as pltpu
from jax.experimental.pallas import tpu_sc as plsc

H = 64
B = 1024
E_ITEM = 51200
E_SOC = 20480
NC = 2
NS = 16
NW = NC * NS
GC = 80
YW = 2 * H


def _mesh():
    return plsc.VectorSubcoreMesh(core_axis_name="c", subcore_axis_name="s")


_SC_PARAMS = pltpu.CompilerParams()


def _worker_id():
    return lax.axis_index("s") * NC + lax.axis_index("c")


NBUF = 4


def _gather_task(table, idx, out, n, idx_all, bufs, sems, w):
    D = table.shape[1]
    per = n // NW
    base = pl.multiple_of(w * per, 8)
    cols = pl.ds(0, D)
    pltpu.sync_copy(idx.at[pl.ds(base, per)], idx_all.at[pl.ds(0, per)])
    if per >= GC * NBUF:
        def body(i, carry):
            c0 = i * NBUF
            hs = [
                pltpu.async_copy(
                    table.at[idx_all.at[pl.ds((c0 + j) * GC, GC)]],
                    bufs[j], sems[j])
                for j in range(NBUF)
            ]
            for j in range(NBUF):
                hs[j].wait()
                pltpu.sync_copy(
                    bufs[j], out.at[pl.ds(base + (c0 + j) * GC, GC), cols])
            return carry
        lax.fori_loop(0, per // (GC * NBUF), body, 0)
    else:
        sl = pl.ds(0, per)
        pltpu.async_copy(table.at[idx_all.at[sl]], bufs[0].at[sl],
                         sems[0]).wait()
        pltpu.sync_copy(bufs[0].at[sl], out.at[pl.ds(base, per), cols])


def _sc_gather_stage1(user_table, item_table, user, batch_target, item,
                      social, batch_i_users):
    outs = [
        jax.ShapeDtypeStruct((B, YW), jnp.float32),
        jax.ShapeDtypeStruct((B, YW), jnp.float32),
        jax.ShapeDtypeStruct((E_ITEM, YW), jnp.float32),
        jax.ShapeDtypeStruct((E_SOC, YW), jnp.float32),
        jax.ShapeDtypeStruct((E_ITEM, YW), jnp.float32),
    ]

    @functools.partial(
        pl.kernel, out_type=outs, mesh=_mesh(),
        compiler_params=_SC_PARAMS,
        scratch_types=[pltpu.VMEM((E_ITEM // NW,), jnp.int32)]
                      + [pltpu.VMEM((GC, YW), jnp.float32)] * NBUF
                      + [pltpu.SemaphoreType.DMA] * NBUF)
    def k(user_t, item_t, user_i, btgt_i, item_i, social_i, biu_i,
          user_emb, tgt, item_emb, social_emb, item_u_emb,
          idx_all, b0, b1, b2, b3, s0, s1, s2, s3):
        w = _worker_id()
        bufs, sems = [b0, b1, b2, b3], [s0, s1, s2, s3]
        _gather_task(user_t, user_i, user_emb, B, idx_all, bufs, sems, w)
        _gather_task(item_t, btgt_i, tgt, B, idx_all, bufs, sems, w)
        _gather_task(item_t, item_i, item_emb, E_ITEM, idx_all, bufs, sems, w)
        _gather_task(user_t, social_i, social_emb, E_SOC, idx_all, bufs, sems,
                     w)
        _gather_task(user_t, biu_i, item_u_emb, E_ITEM, idx_all, bufs, sems,
                     w)

    return k(user_table, item_table, user, batch_target, item, social,
             batch_i_users)


def _sc_gather_stage2(user_emb, tgt, item4user, social4user, batch_u_item):
    outs = [
        jax.ShapeDtypeStruct((E_ITEM, YW), jnp.float32),
        jax.ShapeDtypeStruct((E_SOC, YW), jnp.float32),
        jax.ShapeDtypeStruct((E_ITEM, YW), jnp.float32),
    ]

    @functools.partial(
        pl.kernel, out_type=outs, mesh=_mesh(),
        compiler_params=_SC_PARAMS,
        scratch_types=[pltpu.VMEM((E_ITEM // NW,), jnp.int32)]
                      + [pltpu.VMEM((GC, YW), jnp.float32)] * NBUF
                      + [pltpu.SemaphoreType.DMA] * NBUF)
    def k(ue_t, tgt_t, i4u_i, s4u_i, bui_i, ue_g, su, tgt_g,
          idx_all, b0, b1, b2, b3, s0, s1, s2, s3):
        w = _worker_id()
        bufs, sems = [b0, b1, b2, b3], [s0, s1, s2, s3]
        _gather_task(ue_t, i4u_i, ue_g, E_ITEM, idx_all, bufs, sems, w)
        _gather_task(ue_t, s4u_i, su, E_SOC, idx_all, bufs, sems, w)
        _gather_task(tgt_t, bui_i, tgt_g, E_ITEM, idx_all, bufs, sems, w)

    return k(user_emb, tgt, item4user, social4user, batch_u_item)


CE = 2048


def _tc_edges_proj(x1, rids3, xg, r_table, w_pre, b_pre, w0, b0, w1, b1, n):
    grid = n // CE

    def body(x1_ref, rid_ref, xg_ref, rt_ref, wpre_ref, bpre_ref,
             w0_ref, b0_ref, w1_ref, b1_ref, y_ref):
        ids = rid_ref[0, 0, :]
        oh = (ids[:, None] == lax.broadcasted_iota(jnp.int32, (CE, 16), 1)
              ).astype(jnp.float32)
        x2 = oh @ rt_ref[...]
        wpre = wpre_ref[...]
        xia = (x1_ref[:, :H] @ wpre[:, :H].T + x2 @ wpre[:, H:].T
               + bpre_ref[...])
        w0m = w0_ref[...]
        a1 = jnp.maximum(
            xia @ w0m[:, :H].T + xg_ref[:, :H] @ w0m[:, H:].T + b0_ref[...],
            0.0)
        l = a1 @ w1_ref[...].T + b1_ref[...]
        e = jnp.exp(l)
        y_ref[...] = jnp.concatenate([xia * e, e], axis=1)

    rt16 = jnp.zeros((16, H), jnp.float32).at[:10].set(r_table)
    return pl.pallas_call(
        body,
        grid=(grid,),
        in_specs=[
            pl.BlockSpec((CE, YW), lambda i: (i, 0)),
            pl.BlockSpec((1, 1, CE), lambda i: (i, 0, 0)),
            pl.BlockSpec((CE, YW), lambda i: (i, 0)),
            pl.BlockSpec((16, H), lambda i: (0, 0)),
            pl.BlockSpec((H, 2 * H), lambda i: (0, 0)),
            pl.BlockSpec((1, H), lambda i: (0, 0)),
            pl.BlockSpec((H, 2 * H), lambda i: (0, 0)),
            pl.BlockSpec((1, H), lambda i: (0, 0)),
            pl.BlockSpec((H, H), lambda i: (0, 0)),
            pl.BlockSpec((1, 1), lambda i: (0, 0)),
        ],
        out_specs=pl.BlockSpec((CE, YW), lambda i: (i, 0)),
        out_shape=jax.ShapeDtypeStruct((n, YW), jnp.float32),
    )(x1, rids3, xg, rt16, w_pre, b_pre, w0, b0, w1, b1)


def _tc_edges_soc(se, su, w0, b0, w1, b1):
    grid = E_SOC // CE

    def body(se_ref, su_ref, w0_ref, b0_ref, w1_ref, b1_ref, y_ref):
        su_x = su_ref[:, :H]
        w0m = w0_ref[...]
        a1 = jnp.maximum(
            se_ref[:, :H] @ w0m[:, :H].T + su_x @ w0m[:, H:].T + b0_ref[...],
            0.0)
        l = a1 @ w1_ref[...].T + b1_ref[...]
        e = jnp.exp(l)
        y_ref[...] = jnp.concatenate([su_x * e, e], axis=1)

    return pl.pallas_call(
        body,
        grid=(grid,),
        in_specs=[
            pl.BlockSpec((CE, YW), lambda i: (i, 0)),
            pl.BlockSpec((CE, YW), lambda i: (i, 0)),
            pl.BlockSpec((H, 2 * H), lambda i: (0, 0)),
            pl.BlockSpec((1, H), lambda i: (0, 0)),
            pl.BlockSpec((H, H), lambda i: (0, 0)),
            pl.BlockSpec((1, 1), lambda i: (0, 0)),
        ],
        out_specs=pl.BlockSpec((CE, YW), lambda i: (i, 0)),
        out_shape=jax.ShapeDtypeStruct((E_SOC, YW), jnp.float32),
    )(se, su, w0, b0, w1, b1)


def _sc_segsum(y_i, seg_i, y_u, seg_u, y_s, seg_s, zero_rows):
    outs = [jax.ShapeDtypeStruct((NC, B, YW), jnp.float32)] * 3

    @functools.partial(
        pl.kernel, out_type=outs, mesh=_mesh(),
        compiler_params=_SC_PARAMS,
        scratch_types=[pltpu.VMEM((GC, YW), jnp.float32),
                       pltpu.VMEM((GC, YW), jnp.float32),
                       pltpu.VMEM((GC,), jnp.int32),
                       pltpu.VMEM((GC,), jnp.int32),
                       pltpu.SemaphoreType.DMA,
                       pltpu.SemaphoreType.DMA,
                       pltpu.VMEM_SHARED((B, YW), jnp.float32),
                       pltpu.VMEM_SHARED((B, YW), jnp.float32),
                       pltpu.VMEM_SHARED((B, YW), jnp.float32)])
    def k(yi, si, yu, su_, ys, ss, z, out_i, out_u, out_s,
          y_v, y_v1, seg_v, seg_v1, ysem0, ysem1, acc_i, acc_u, acc_s):
        c = lax.axis_index("c")
        s = lax.axis_index("s")
        w = s * NC + c

        @pl.when(s == 0)
        def _zero():
            pltpu.sync_copy(z, acc_i)
            pltpu.sync_copy(z, acc_u)
            pltpu.sync_copy(z, acc_s)

        plsc.subcore_barrier()

        def task(y, seg, acc, n):
            per = n // NW
            base = pl.multiple_of(w * per, 8)

            def body(i, carry):
                o0 = pl.multiple_of(base + (2 * i) * GC, 8)
                o1 = pl.multiple_of(base + (2 * i + 1) * GC, 8)
                h0 = pltpu.async_copy(y.at[pl.ds(o0, GC)], y_v, ysem0)
                h1 = pltpu.async_copy(y.at[pl.ds(o1, GC)], y_v1, ysem1)
                pltpu.sync_copy(seg.at[pl.ds(o0, GC)], seg_v)
                pltpu.sync_copy(seg.at[pl.ds(o1, GC)], seg_v1)
                h0.wait()
                pltpu.sync_copy(y_v, acc.at[seg_v], add=True)
                h1.wait()
                pltpu.sync_copy(y_v1, acc.at[seg_v1], add=True)
                return carry

            lax.fori_loop(0, per // (2 * GC), body, 0)

        task(yi, si, acc_i, E_ITEM)
        task(yu, su_, acc_u, E_ITEM)
        task(ys, ss, acc_s, E_SOC)

        plsc.subcore_barrier()
        rows = B // NS
        sl = pl.ds(s * rows, rows)
        pltpu.sync_copy(acc_i.at[sl], out_i.at[c, sl])
        pltpu.sync_copy(acc_u.at[sl], out_u.at[c, sl])
        pltpu.sync_copy(acc_s.at[sl], out_s.at[c, sl])

    return k(y_i, seg_i, y_u, seg_u, y_s, seg_s, zero_rows)


CI = 2048


def _tc_final(agg_i, agg_u, agg_s, item_table,
              eq4_w, eq4_b, eq4i_w, eq4i_b, eq9_w, eq9_b, eq13_w, eq13_b,
              mlp_w0, mlp_b0, mlp_w1, mlp_b1, mlp_w2, mlp_b2):
    grid = pl.cdiv(item_table.shape[0], CI)

    def body(ai, au, asoc, it, e4w, e4b, e4iw, e4ib, e9w, e9b,
             e13w, e13b, m0w, m0b, m1w, m1b, m2w, m2b,
             m_ref, sc_ref, h_scr):
        pid = pl.program_id(0)

        @pl.when(pid == 0)
        def _head():
            def head(agg, wt, bt):
                a = agg[0] + agg[1]
                v = a[:, :H]
                sden = a[:, H:]
                return jnp.maximum((v / (sden + 1e-16)) @ wt[...].T + bt[...],
                                   0.0)

            hi = head(ai[...], e4w, e4b)
            zj = head(au[...], e4iw, e4ib)
            hs = head(asoc[...], e9w, e9b)
            e13 = e13w[...]
            h = jnp.maximum(hi @ e13[:, :H].T + hs @ e13[:, H:].T + e13b[...],
                            0.0)
            h_scr[...] = h
            m0 = m0w[...]
            mm = h @ m0[:, :H].T + zj @ m0[:, H:].T + m0b[...]
            mm = jnp.maximum(mm, 0.0) @ m1w[...].T + m1b[...]
            mm = jnp.maximum(mm, 0.0) @ m2w[...].T + m2b[...]
            m_ref[...] = mm[:, :1]

        sc_ref[...] = lax.dot_general(it[...], h_scr[...],
                                      (((1,), (1,)), ((), ())))

    full = lambda i: (0, 0)
    full3 = lambda i: (0, 0, 0)
    return pl.pallas_call(
        body,
        grid=(grid,),
        in_specs=[
            pl.BlockSpec((NC, B, YW), full3),
            pl.BlockSpec((NC, B, YW), full3),
            pl.BlockSpec((NC, B, YW), full3),
            pl.BlockSpec((CI, H), lambda i: (i, 0)),
            pl.BlockSpec((H, H), full),
            pl.BlockSpec((1, H), full),
            pl.BlockSpec((H, H), full),
            pl.BlockSpec((1, H), full),
            pl.BlockSpec((H, H), full),
            pl.BlockSpec((1, H), full),
            pl.BlockSpec((H, 2 * H), full),
            pl.BlockSpec((1, H), full),
            pl.BlockSpec((H, 2 * H), full),
            pl.BlockSpec((1, H), full),
            pl.BlockSpec((H, H), full),
            pl.BlockSpec((1, H), full),
            pl.BlockSpec((H, H), full),
            pl.BlockSpec((1, 1), full),
        ],
        out_specs=[
            pl.BlockSpec((B, 1), lambda i: (0, 0)),
            pl.BlockSpec((CI, B), lambda i: (i, 0)),
        ],
        out_shape=[
            jax.ShapeDtypeStruct((B, 1), jnp.float32),
            jax.ShapeDtypeStruct((item_table.shape[0], B), jnp.float32),
        ],
        scratch_shapes=[pltpu.VMEM((B, H), jnp.float32)],
    )(agg_i, agg_u, agg_s, item_table,
      eq4_w, eq4_b, eq4i_w, eq4i_b, eq9_w, eq9_b, eq13_w, eq13_b,
      mlp_w0, mlp_b0, mlp_w1, mlp_b1, mlp_w2, mlp_b2)


def kernel(user, item, rating, item4user, social, social4user,
           batch_i_users, batch_i_ratings, batch_u_item, batch_target,
           user_table, item_table, rating_table,
           gv_w, gv_b, gu_w, gu_b,
           eq5_w0, eq5_b0, eq5_w1, eq5_b1,
           eq5i_w0, eq5i_b0, eq5i_w1, eq5i_b1,
           eq4_w, eq4_b, eq4i_w, eq4i_b,
           eq10_w0, eq10_b0, eq10_w1, eq10_b1,
           eq9_w, eq9_b, eq13_w, eq13_b,
           mlp_w0, mlp_b0, mlp_w1, mlp_b1, mlp_w2, mlp_b2):
    i32 = jnp.int32
    user = user.astype(i32)
    item = item.astype(i32)
    rating = rating.astype(i32)
    item4user = item4user.astype(i32)
    social = social.astype(i32)
    social4user = social4user.astype(i32)
    batch_i_users = batch_i_users.astype(i32)
    batch_i_ratings = batch_i_ratings.astype(i32)
    batch_u_item = batch_u_item.astype(i32)
    batch_target = batch_target.astype(i32)

    ut128 = jnp.pad(user_table, ((0, 7), (0, H)))
    it128 = jnp.pad(item_table, ((0, 0), (0, H)))
    user_emb, tgt, item_emb, social_emb, item_u_emb = _sc_gather_stage1(
        ut128, it128, user, batch_target, item, social,
        batch_i_users)
    ue_g, su, tgt_g = _sc_gather_stage2(
        user_emb, tgt, item4user, social4user, batch_u_item)

    r2 = lambda b: b.reshape(1, -1)
    rep = lambda w: jnp.broadcast_to(w, (H, H))
    rids3 = rating.reshape(E_ITEM // CE, 1, CE)
    brids3 = batch_i_ratings.reshape(E_ITEM // CE, 1, CE)
    y_i = _tc_edges_proj(item_emb, rids3, ue_g, rating_table,
                         gv_w, r2(gv_b), eq5_w0, r2(eq5_b0), rep(eq5_w1),
                         r2(eq5_b1), E_ITEM)
    y_u = _tc_edges_proj(item_u_emb, brids3, tgt_g, rating_table,
                         gu_w, r2(gu_b), eq5i_w0, r2(eq5i_b0), rep(eq5i_w1),
                         r2(eq5i_b1), E_ITEM)
    y_s = _tc_edges_soc(social_emb, su, eq10_w0, r2(eq10_b0), rep(eq10_w1),
                        r2(eq10_b1))

    zero_rows = jnp.zeros((B, YW), jnp.float32)
    agg_i, agg_u, agg_s = _sc_segsum(y_i, item4user, y_u, batch_u_item,
                                     y_s, social4user, zero_rows)

    m, scores_t = _tc_final(agg_i, agg_u, agg_s, item_table,
                            eq4_w, r2(eq4_b), eq4i_w, r2(eq4i_b),
                            eq9_w, r2(eq9_b), eq13_w, r2(eq13_b),
                            mlp_w0, r2(mlp_b0), mlp_w1, r2(mlp_b1),
                            rep(mlp_w2), mlp_b2.reshape(1, 1))
    return m, scores_t.T


# speedup vs baseline: 1.9977x; 1.3289x over previous
import functools

import jax
import jax.numpy as jnp
from jax import lax
from jax.experimental import pallas as pl
from jax.experimental.pallas import tpu as pltpu
from jax.experimental.pallas import tpu_sc as plsc

H = 64
B = 1024
E_ITEM = 51200
E_SOC = 20480
NC = 2
NS = 16
NW = NC * NS
GC = 80
YW = 2 * H


def _mesh():
    return plsc.VectorSubcoreMesh(core_axis_name="c", subcore_axis_name="s")


_SC_PARAMS = pltpu.CompilerParams()


def _worker_id():
    return lax.axis_index("s") * NC + lax.axis_index("c")


NBUF = 4


def _gather_task(table, idx, out, n, idx_all, bufs, sems, w):
    D = table.shape[1]
    per = n // NW
    base = pl.multiple_of(w * per, 8)
    cols = pl.ds(0, D)
    pltpu.sync_copy(idx.at[pl.ds(base, per)], idx_all.at[pl.ds(0, per)])
    if per >= GC * NBUF:
        def body(i, carry):
            c0 = i * NBUF
            hs = [
                pltpu.async_copy(
                    table.at[idx_all.at[pl.ds((c0 + j) * GC, GC)]],
                    bufs[j], sems[j])
                for j in range(NBUF)
            ]
            for j in range(NBUF):
                hs[j].wait()
                pltpu.sync_copy(
                    bufs[j], out.at[pl.ds(base + (c0 + j) * GC, GC), cols])
            return carry
        lax.fori_loop(0, per // (GC * NBUF), body, 0)
    else:
        sl = pl.ds(0, per)
        pltpu.async_copy(table.at[idx_all.at[sl]], bufs[0].at[sl],
                         sems[0]).wait()
        pltpu.sync_copy(bufs[0].at[sl], out.at[pl.ds(base, per), cols])


def _sc_gather_stage1(user_table, item_table, user, batch_target, item,
                      social, batch_i_users):
    outs = [
        jax.ShapeDtypeStruct((B, YW), jnp.float32),
        jax.ShapeDtypeStruct((B, YW), jnp.float32),
        jax.ShapeDtypeStruct((E_ITEM, YW), jnp.float32),
        jax.ShapeDtypeStruct((E_SOC, YW), jnp.float32),
        jax.ShapeDtypeStruct((E_ITEM, YW), jnp.float32),
    ]

    @functools.partial(
        pl.kernel, out_type=outs, mesh=_mesh(),
        compiler_params=_SC_PARAMS,
        scratch_types=[pltpu.VMEM((E_ITEM // NW,), jnp.int32)]
                      + [pltpu.VMEM((GC, YW), jnp.float32)] * NBUF
                      + [pltpu.SemaphoreType.DMA] * NBUF)
    def k(user_t, item_t, user_i, btgt_i, item_i, social_i, biu_i,
          user_emb, tgt, item_emb, social_emb, item_u_emb,
          idx_all, b0, b1, b2, b3, s0, s1, s2, s3):
        w = _worker_id()
        bufs, sems = [b0, b1, b2, b3], [s0, s1, s2, s3]
        _gather_task(user_t, user_i, user_emb, B, idx_all, bufs, sems, w)
        _gather_task(item_t, btgt_i, tgt, B, idx_all, bufs, sems, w)
        _gather_task(item_t, item_i, item_emb, E_ITEM, idx_all, bufs, sems, w)
        _gather_task(user_t, social_i, social_emb, E_SOC, idx_all, bufs, sems,
                     w)
        _gather_task(user_t, biu_i, item_u_emb, E_ITEM, idx_all, bufs, sems,
                     w)

    return k(user_table, item_table, user, batch_target, item, social,
             batch_i_users)


def _sc_gather_stage2(user_emb, tgt, item4user, social4user, batch_u_item):
    outs = [
        jax.ShapeDtypeStruct((E_ITEM, YW), jnp.float32),
        jax.ShapeDtypeStruct((E_SOC, YW), jnp.float32),
        jax.ShapeDtypeStruct((E_ITEM, YW), jnp.float32),
    ]

    @functools.partial(
        pl.kernel, out_type=outs, mesh=_mesh(),
        compiler_params=_SC_PARAMS,
        scratch_types=[pltpu.VMEM((E_ITEM // NW,), jnp.int32)]
                      + [pltpu.VMEM((GC, YW), jnp.float32)] * NBUF
                      + [pltpu.SemaphoreType.DMA] * NBUF)
    def k(ue_t, tgt_t, i4u_i, s4u_i, bui_i, ue_g, su, tgt_g,
          idx_all, b0, b1, b2, b3, s0, s1, s2, s3):
        w = _worker_id()
        bufs, sems = [b0, b1, b2, b3], [s0, s1, s2, s3]
        _gather_task(ue_t, i4u_i, ue_g, E_ITEM, idx_all, bufs, sems, w)
        _gather_task(ue_t, s4u_i, su, E_SOC, idx_all, bufs, sems, w)
        _gather_task(tgt_t, bui_i, tgt_g, E_ITEM, idx_all, bufs, sems, w)

    return k(user_emb, tgt, item4user, social4user, batch_u_item)


CB = 8192


def _tc_pad_table(table):
    t_t = table.T
    n = table.shape[0]
    grid = pl.cdiv(n, CB)
    eye = jnp.eye(H, dtype=jnp.float32)

    def body(t_ref, e_ref, o_ref):
        o_ref[:, :H] = lax.dot_general(t_ref[...], e_ref[...],
                                       (((0,), (0,)), ((), ())))

    return pl.pallas_call(
        body,
        grid=(grid,),
        in_specs=[pl.BlockSpec((H, CB), lambda i: (0, i)),
                  pl.BlockSpec((H, H), lambda i: (0, 0))],
        out_specs=pl.BlockSpec((CB, YW), lambda i: (i, 0)),
        out_shape=jax.ShapeDtypeStruct((n, YW), jnp.float32),
    )(t_t, eye)


CE = 2048


def _tc_edges_proj(x1, rids3, xg, r_table, w_pre, b_pre, w0, b0, w1, b1, n):
    grid = n // CE

    def body(x1_ref, rid_ref, xg_ref, rt_ref, wpre_ref, bpre_ref,
             w0_ref, b0_ref, w1_ref, b1_ref, y_ref):
        ids = rid_ref[0, 0, :]
        oh = (ids[:, None] == lax.broadcasted_iota(jnp.int32, (CE, 16), 1)
              ).astype(jnp.float32)
        x2 = oh @ rt_ref[...]
        wpre = wpre_ref[...]
        xia = (x1_ref[:, :H] @ wpre[:, :H].T + x2 @ wpre[:, H:].T
               + bpre_ref[...])
        w0m = w0_ref[...]
        a1 = jnp.maximum(
            xia @ w0m[:, :H].T + xg_ref[:, :H] @ w0m[:, H:].T + b0_ref[...],
            0.0)
        l = a1 @ w1_ref[...].T + b1_ref[...]
        e = jnp.exp(l)
        y_ref[...] = jnp.concatenate([xia * e, e], axis=1)

    rt16 = jnp.zeros((16, H), jnp.float32).at[:10].set(r_table)
    return pl.pallas_call(
        body,
        grid=(grid,),
        in_specs=[
            pl.BlockSpec((CE, YW), lambda i: (i, 0)),
            pl.BlockSpec((1, 1, CE), lambda i: (i, 0, 0)),
            pl.BlockSpec((CE, YW), lambda i: (i, 0)),
            pl.BlockSpec((16, H), lambda i: (0, 0)),
            pl.BlockSpec((H, 2 * H), lambda i: (0, 0)),
            pl.BlockSpec((1, H), lambda i: (0, 0)),
            pl.BlockSpec((H, 2 * H), lambda i: (0, 0)),
            pl.BlockSpec((1, H), lambda i: (0, 0)),
            pl.BlockSpec((H, H), lambda i: (0, 0)),
            pl.BlockSpec((1, 1), lambda i: (0, 0)),
        ],
        out_specs=pl.BlockSpec((CE, YW), lambda i: (i, 0)),
        out_shape=jax.ShapeDtypeStruct((n, YW), jnp.float32),
    )(x1, rids3, xg, rt16, w_pre, b_pre, w0, b0, w1, b1)


def _tc_edges_soc(se, su, w0, b0, w1, b1):
    grid = E_SOC // CE

    def body(se_ref, su_ref, w0_ref, b0_ref, w1_ref, b1_ref, y_ref):
        su_x = su_ref[:, :H]
        w0m = w0_ref[...]
        a1 = jnp.maximum(
            se_ref[:, :H] @ w0m[:, :H].T + su_x @ w0m[:, H:].T + b0_ref[...],
            0.0)
        l = a1 @ w1_ref[...].T + b1_ref[...]
        e = jnp.exp(l)
        y_ref[...] = jnp.concatenate([su_x * e, e], axis=1)

    return pl.pallas_call(
        body,
        grid=(grid,),
        in_specs=[
            pl.BlockSpec((CE, YW), lambda i: (i, 0)),
            pl.BlockSpec((CE, YW), lambda i: (i, 0)),
            pl.BlockSpec((H, 2 * H), lambda i: (0, 0)),
            pl.BlockSpec((1, H), lambda i: (0, 0)),
            pl.BlockSpec((H, H), lambda i: (0, 0)),
            pl.BlockSpec((1, 1), lambda i: (0, 0)),
        ],
        out_specs=pl.BlockSpec((CE, YW), lambda i: (i, 0)),
        out_shape=jax.ShapeDtypeStruct((E_SOC, YW), jnp.float32),
    )(se, su, w0, b0, w1, b1)


def _sc_segsum(y_i, seg_i, y_u, seg_u, y_s, seg_s, zero_rows):
    outs = [jax.ShapeDtypeStruct((NC, B, YW), jnp.float32)] * 3

    @functools.partial(
        pl.kernel, out_type=outs, mesh=_mesh(),
        compiler_params=_SC_PARAMS,
        scratch_types=[pltpu.VMEM((GC, YW), jnp.float32),
                       pltpu.VMEM((GC, YW), jnp.float32),
                       pltpu.VMEM((GC,), jnp.int32),
                       pltpu.VMEM((GC,), jnp.int32),
                       pltpu.SemaphoreType.DMA,
                       pltpu.SemaphoreType.DMA,
                       pltpu.VMEM_SHARED((B, YW), jnp.float32),
                       pltpu.VMEM_SHARED((B, YW), jnp.float32),
                       pltpu.VMEM_SHARED((B, YW), jnp.float32)])
    def k(yi, si, yu, su_, ys, ss, z, out_i, out_u, out_s,
          y_v, y_v1, seg_v, seg_v1, ysem0, ysem1, acc_i, acc_u, acc_s):
        c = lax.axis_index("c")
        s = lax.axis_index("s")
        w = s * NC + c

        @pl.when(s == 0)
        def _zero():
            pltpu.sync_copy(z, acc_i)
            pltpu.sync_copy(z, acc_u)
            pltpu.sync_copy(z, acc_s)

        plsc.subcore_barrier()

        def task(y, seg, acc, n):
            per = n // NW
            base = pl.multiple_of(w * per, 8)

            def body(i, carry):
                o0 = pl.multiple_of(base + (2 * i) * GC, 8)
                o1 = pl.multiple_of(base + (2 * i + 1) * GC, 8)
                h0 = pltpu.async_copy(y.at[pl.ds(o0, GC)], y_v, ysem0)
                h1 = pltpu.async_copy(y.at[pl.ds(o1, GC)], y_v1, ysem1)
                pltpu.sync_copy(seg.at[pl.ds(o0, GC)], seg_v)
                pltpu.sync_copy(seg.at[pl.ds(o1, GC)], seg_v1)
                h0.wait()
                pltpu.sync_copy(y_v, acc.at[seg_v], add=True)
                h1.wait()
                pltpu.sync_copy(y_v1, acc.at[seg_v1], add=True)
                return carry

            lax.fori_loop(0, per // (2 * GC), body, 0)

        task(yi, si, acc_i, E_ITEM)
        task(yu, su_, acc_u, E_ITEM)
        task(ys, ss, acc_s, E_SOC)

        plsc.subcore_barrier()
        rows = B // NS
        sl = pl.ds(s * rows, rows)
        pltpu.sync_copy(acc_i.at[sl], out_i.at[c, sl])
        pltpu.sync_copy(acc_u.at[sl], out_u.at[c, sl])
        pltpu.sync_copy(acc_s.at[sl], out_s.at[c, sl])

    return k(y_i, seg_i, y_u, seg_u, y_s, seg_s, zero_rows)


CI = 2048


def _tc_final(agg_i, agg_u, agg_s, item_table,
              eq4_w, eq4_b, eq4i_w, eq4i_b, eq9_w, eq9_b, eq13_w, eq13_b,
              mlp_w0, mlp_b0, mlp_w1, mlp_b1, mlp_w2, mlp_b2):
    grid = pl.cdiv(item_table.shape[0], CI)

    def body(ai, au, asoc, it, e4w, e4b, e4iw, e4ib, e9w, e9b,
             e13w, e13b, m0w, m0b, m1w, m1b, m2w, m2b,
             m_ref, sc_ref, h_scr):
        pid = pl.program_id(0)

        @pl.when(pid == 0)
        def _head():
            def head(agg, wt, bt):
                a = agg[0] + agg[1]
                v = a[:, :H]
                sden = a[:, H:]
                return jnp.maximum((v / (sden + 1e-16)) @ wt[...].T + bt[...],
                                   0.0)

            hi = head(ai[...], e4w, e4b)
            zj = head(au[...], e4iw, e4ib)
            hs = head(asoc[...], e9w, e9b)
            e13 = e13w[...]
            h = jnp.maximum(hi @ e13[:, :H].T + hs @ e13[:, H:].T + e13b[...],
                            0.0)
            h_scr[...] = h
            m0 = m0w[...]
            mm = h @ m0[:, :H].T + zj @ m0[:, H:].T + m0b[...]
            mm = jnp.maximum(mm, 0.0) @ m1w[...].T + m1b[...]
            mm = jnp.maximum(mm, 0.0) @ m2w[...].T + m2b[...]
            m_ref[...] = mm[:, :1]

        sc_ref[...] = lax.dot_general(it[:, :H], h_scr[...],
                                      (((1,), (1,)), ((), ())))

    full = lambda i: (0, 0)
    full3 = lambda i: (0, 0, 0)
    return pl.pallas_call(
        body,
        grid=(grid,),
        in_specs=[
            pl.BlockSpec((NC, B, YW), full3),
            pl.BlockSpec((NC, B, YW), full3),
            pl.BlockSpec((NC, B, YW), full3),
            pl.BlockSpec((CI, YW), lambda i: (i, 0)),
            pl.BlockSpec((H, H), full),
            pl.BlockSpec((1, H), full),
            pl.BlockSpec((H, H), full),
            pl.BlockSpec((1, H), full),
            pl.BlockSpec((H, H), full),
            pl.BlockSpec((1, H), full),
            pl.BlockSpec((H, 2 * H), full),
            pl.BlockSpec((1, H), full),
            pl.BlockSpec((H, 2 * H), full),
            pl.BlockSpec((1, H), full),
            pl.BlockSpec((H, H), full),
            pl.BlockSpec((1, H), full),
            pl.BlockSpec((H, H), full),
            pl.BlockSpec((1, 1), full),
        ],
        out_specs=[
            pl.BlockSpec((B, 1), lambda i: (0, 0)),
            pl.BlockSpec((CI, B), lambda i: (i, 0)),
        ],
        out_shape=[
            jax.ShapeDtypeStruct((B, 1), jnp.float32),
            jax.ShapeDtypeStruct((item_table.shape[0], B), jnp.float32),
        ],
        scratch_shapes=[pltpu.VMEM((B, H), jnp.float32)],
    )(agg_i, agg_u, agg_s, item_table,
      eq4_w, eq4_b, eq4i_w, eq4i_b, eq9_w, eq9_b, eq13_w, eq13_b,
      mlp_w0, mlp_b0, mlp_w1, mlp_b1, mlp_w2, mlp_b2)


def kernel(user, item, rating, item4user, social, social4user,
           batch_i_users, batch_i_ratings, batch_u_item, batch_target,
           user_table, item_table, rating_table,
           gv_w, gv_b, gu_w, gu_b,
           eq5_w0, eq5_b0, eq5_w1, eq5_b1,
           eq5i_w0, eq5i_b0, eq5i_w1, eq5i_b1,
           eq4_w, eq4_b, eq4i_w, eq4i_b,
           eq10_w0, eq10_b0, eq10_w1, eq10_b1,
           eq9_w, eq9_b, eq13_w, eq13_b,
           mlp_w0, mlp_b0, mlp_w1, mlp_b1, mlp_w2, mlp_b2):
    i32 = jnp.int32
    user = user.astype(i32)
    item = item.astype(i32)
    rating = rating.astype(i32)
    item4user = item4user.astype(i32)
    social = social.astype(i32)
    social4user = social4user.astype(i32)
    batch_i_users = batch_i_users.astype(i32)
    batch_i_ratings = batch_i_ratings.astype(i32)
    batch_u_item = batch_u_item.astype(i32)
    batch_target = batch_target.astype(i32)

    ut128 = _tc_pad_table(user_table)
    it128 = _tc_pad_table(item_table)
    user_emb, tgt, item_emb, social_emb, item_u_emb = _sc_gather_stage1(
        ut128, it128, user, batch_target, item, social,
        batch_i_users)
    ue_g, su, tgt_g = _sc_gather_stage2(
        user_emb, tgt, item4user, social4user, batch_u_item)

    r2 = lambda b: b.reshape(1, -1)
    rep = lambda w: jnp.broadcast_to(w, (H, H))
    rids3 = rating.reshape(E_ITEM // CE, 1, CE)
    brids3 = batch_i_ratings.reshape(E_ITEM // CE, 1, CE)
    y_i = _tc_edges_proj(item_emb, rids3, ue_g, rating_table,
                         gv_w, r2(gv_b), eq5_w0, r2(eq5_b0), rep(eq5_w1),
                         r2(eq5_b1), E_ITEM)
    y_u = _tc_edges_proj(item_u_emb, brids3, tgt_g, rating_table,
                         gu_w, r2(gu_b), eq5i_w0, r2(eq5i_b0), rep(eq5i_w1),
                         r2(eq5i_b1), E_ITEM)
    y_s = _tc_edges_soc(social_emb, su, eq10_w0, r2(eq10_b0), rep(eq10_w1),
                        r2(eq10_b1))

    zero_rows = jnp.zeros((B, YW), jnp.float32)
    agg_i, agg_u, agg_s = _sc_segsum(y_i, item4user, y_u, batch_u_item,
                                     y_s, social4user, zero_rows)

    m, scores_t = _tc_final(agg_i, agg_u, agg_s, it128,
                            eq4_w, r2(eq4_b), eq4i_w, r2(eq4i_b),
                            eq9_w, r2(eq9_b), eq13_w, r2(eq13_b),
                            mlp_w0, r2(mlp_b0), mlp_w1, r2(mlp_b1),
                            rep(mlp_w2), mlp_b2.reshape(1, 1))
    return m, scores_t.T


# speedup vs baseline: 2.0226x; 1.0124x over previous
import functools

import jax
import jax.numpy as jnp
from jax import lax
from jax.experimental import pallas as pl
from jax.experimental.pallas import tpu as pltpu
from jax.experimental.pallas import tpu_sc as plsc

H = 64
B = 1024
E_ITEM = 51200
E_SOC = 20480
NC = 2
NS = 16
NW = NC * NS
GC = 80
YW = 2 * H


def _mesh():
    return plsc.VectorSubcoreMesh(core_axis_name="c", subcore_axis_name="s")


_SC_PARAMS = pltpu.CompilerParams()


def _worker_id():
    return lax.axis_index("s") * NC + lax.axis_index("c")


NBUF = 4


def _gather_task(table, idx, out, n, idx_all, bufs, sems, w):
    D = table.shape[1]
    per = n // NW
    base = pl.multiple_of(w * per, 8)
    cols = pl.ds(0, D)
    pltpu.sync_copy(idx.at[pl.ds(base, per)], idx_all.at[pl.ds(0, per)])
    if per >= GC * NBUF:
        def body(i, carry):
            c0 = i * NBUF
            hs = [
                pltpu.async_copy(
                    table.at[idx_all.at[pl.ds((c0 + j) * GC, GC)]],
                    bufs[j], sems[j])
                for j in range(NBUF)
            ]
            for j in range(NBUF):
                hs[j].wait()
                pltpu.sync_copy(
                    bufs[j], out.at[pl.ds(base + (c0 + j) * GC, GC), cols])
            return carry
        lax.fori_loop(0, per // (GC * NBUF), body, 0)
    else:
        sl = pl.ds(0, per)
        pltpu.async_copy(table.at[idx_all.at[sl]], bufs[0].at[sl],
                         sems[0]).wait()
        pltpu.sync_copy(bufs[0].at[sl], out.at[pl.ds(base, per), cols])


def _sc_gather_item(item_table, batch_target, item):
    outs = [
        jax.ShapeDtypeStruct((B, YW), jnp.float32),
        jax.ShapeDtypeStruct((E_ITEM, YW), jnp.float32),
    ]

    @functools.partial(
        pl.kernel, out_type=outs, mesh=_mesh(),
        compiler_params=_SC_PARAMS,
        scratch_types=[pltpu.VMEM((E_ITEM // NW,), jnp.int32)]
                      + [pltpu.VMEM((GC, YW), jnp.float32)] * NBUF
                      + [pltpu.SemaphoreType.DMA] * NBUF)
    def k(item_t, btgt_i, item_i, tgt, item_emb,
          idx_all, b0, b1, b2, b3, s0, s1, s2, s3):
        w = _worker_id()
        bufs, sems = [b0, b1, b2, b3], [s0, s1, s2, s3]
        _gather_task(item_t, btgt_i, tgt, B, idx_all, bufs, sems, w)
        _gather_task(item_t, item_i, item_emb, E_ITEM, idx_all, bufs, sems, w)

    return k(item_table, batch_target, item)


def _sc_gather_user(user_table, user, social, batch_i_users):
    outs = [
        jax.ShapeDtypeStruct((B, YW), jnp.float32),
        jax.ShapeDtypeStruct((E_SOC, YW), jnp.float32),
        jax.ShapeDtypeStruct((E_ITEM, YW), jnp.float32),
    ]

    @functools.partial(
        pl.kernel, out_type=outs, mesh=_mesh(),
        compiler_params=_SC_PARAMS,
        scratch_types=[pltpu.VMEM((E_ITEM // NW,), jnp.int32)]
                      + [pltpu.VMEM((GC, YW), jnp.float32)] * NBUF
                      + [pltpu.SemaphoreType.DMA] * NBUF)
    def k(user_t, user_i, social_i, biu_i,
          user_emb, social_emb, item_u_emb,
          idx_all, b0, b1, b2, b3, s0, s1, s2, s3):
        w = _worker_id()
        bufs, sems = [b0, b1, b2, b3], [s0, s1, s2, s3]
        _gather_task(user_t, user_i, user_emb, B, idx_all, bufs, sems, w)
        _gather_task(user_t, social_i, social_emb, E_SOC, idx_all, bufs, sems,
                     w)
        _gather_task(user_t, biu_i, item_u_emb, E_ITEM, idx_all, bufs, sems,
                     w)

    return k(user_table, user, social, batch_i_users)


def _sc_gather_stage2(user_emb, tgt, item4user, social4user, batch_u_item):
    outs = [
        jax.ShapeDtypeStruct((E_ITEM, YW), jnp.float32),
        jax.ShapeDtypeStruct((E_SOC, YW), jnp.float32),
        jax.ShapeDtypeStruct((E_ITEM, YW), jnp.float32),
    ]

    @functools.partial(
        pl.kernel, out_type=outs, mesh=_mesh(),
        compiler_params=_SC_PARAMS,
        scratch_types=[pltpu.VMEM((E_ITEM // NW,), jnp.int32)]
                      + [pltpu.VMEM((GC, YW), jnp.float32)] * NBUF
                      + [pltpu.SemaphoreType.DMA] * NBUF)
    def k(ue_t, tgt_t, i4u_i, s4u_i, bui_i, ue_g, su, tgt_g,
          idx_all, b0, b1, b2, b3, s0, s1, s2, s3):
        w = _worker_id()
        bufs, sems = [b0, b1, b2, b3], [s0, s1, s2, s3]
        _gather_task(ue_t, i4u_i, ue_g, E_ITEM, idx_all, bufs, sems, w)
        _gather_task(ue_t, s4u_i, su, E_SOC, idx_all, bufs, sems, w)
        _gather_task(tgt_t, bui_i, tgt_g, E_ITEM, idx_all, bufs, sems, w)

    return k(user_emb, tgt, item4user, social4user, batch_u_item)


CB = 8192


def _tc_pad_table(table):
    t_t = table.T
    n = table.shape[0]
    grid = pl.cdiv(n, CB)
    eye = jnp.eye(H, dtype=jnp.float32)

    def body(t_ref, e_ref, o_ref):
        o_ref[:, :H] = lax.dot_general(t_ref[...], e_ref[...],
                                       (((0,), (0,)), ((), ())))

    return pl.pallas_call(
        body,
        grid=(grid,),
        in_specs=[pl.BlockSpec((H, CB), lambda i: (0, i)),
                  pl.BlockSpec((H, H), lambda i: (0, 0))],
        out_specs=pl.BlockSpec((CB, YW), lambda i: (i, 0)),
        out_shape=jax.ShapeDtypeStruct((n, YW), jnp.float32),
    )(t_t, eye)


CE = 2048


def _tc_edges_proj(x1, rids3, xg, r_table, w_pre, b_pre, w0, b0, w1, b1, n):
    grid = n // CE

    def body(x1_ref, rid_ref, xg_ref, rt_ref, wpre_ref, bpre_ref,
             w0_ref, b0_ref, w1_ref, b1_ref, y_ref):
        ids = rid_ref[0, 0, :]
        oh = (ids[:, None] == lax.broadcasted_iota(jnp.int32, (CE, 16), 1)
              ).astype(jnp.float32)
        x2 = oh @ rt_ref[...]
        wpre = wpre_ref[...]
        xia = (x1_ref[:, :H] @ wpre[:, :H].T + x2 @ wpre[:, H:].T
               + bpre_ref[...])
        w0m = w0_ref[...]
        a1 = jnp.maximum(
            xia @ w0m[:, :H].T + xg_ref[:, :H] @ w0m[:, H:].T + b0_ref[...],
            0.0)
        l = a1 @ w1_ref[...].T + b1_ref[...]
        e = jnp.exp(l)
        y_ref[...] = jnp.concatenate([xia * e, e], axis=1)

    rt16 = jnp.zeros((16, H), jnp.float32).at[:10].set(r_table)
    return pl.pallas_call(
        body,
        grid=(grid,),
        in_specs=[
            pl.BlockSpec((CE, YW), lambda i: (i, 0)),
            pl.BlockSpec((1, 1, CE), lambda i: (i, 0, 0)),
            pl.BlockSpec((CE, YW), lambda i: (i, 0)),
            pl.BlockSpec((16, H), lambda i: (0, 0)),
            pl.BlockSpec((H, 2 * H), lambda i: (0, 0)),
            pl.BlockSpec((1, H), lambda i: (0, 0)),
            pl.BlockSpec((H, 2 * H), lambda i: (0, 0)),
            pl.BlockSpec((1, H), lambda i: (0, 0)),
            pl.BlockSpec((H, H), lambda i: (0, 0)),
            pl.BlockSpec((1, 1), lambda i: (0, 0)),
        ],
        out_specs=pl.BlockSpec((CE, YW), lambda i: (i, 0)),
        out_shape=jax.ShapeDtypeStruct((n, YW), jnp.float32),
    )(x1, rids3, xg, rt16, w_pre, b_pre, w0, b0, w1, b1)


def _tc_edges_soc(se, su, w0, b0, w1, b1):
    grid = E_SOC // CE

    def body(se_ref, su_ref, w0_ref, b0_ref, w1_ref, b1_ref, y_ref):
        su_x = su_ref[:, :H]
        w0m = w0_ref[...]
        a1 = jnp.maximum(
            se_ref[:, :H] @ w0m[:, :H].T + su_x @ w0m[:, H:].T + b0_ref[...],
            0.0)
        l = a1 @ w1_ref[...].T + b1_ref[...]
        e = jnp.exp(l)
        y_ref[...] = jnp.concatenate([su_x * e, e], axis=1)

    return pl.pallas_call(
        body,
        grid=(grid,),
        in_specs=[
            pl.BlockSpec((CE, YW), lambda i: (i, 0)),
            pl.BlockSpec((CE, YW), lambda i: (i, 0)),
            pl.BlockSpec((H, 2 * H), lambda i: (0, 0)),
            pl.BlockSpec((1, H), lambda i: (0, 0)),
            pl.BlockSpec((H, H), lambda i: (0, 0)),
            pl.BlockSpec((1, 1), lambda i: (0, 0)),
        ],
        out_specs=pl.BlockSpec((CE, YW), lambda i: (i, 0)),
        out_shape=jax.ShapeDtypeStruct((E_SOC, YW), jnp.float32),
    )(se, su, w0, b0, w1, b1)


def _sc_segsum(y_i, seg_i, y_u, seg_u, y_s, seg_s, zero_rows):
    outs = [jax.ShapeDtypeStruct((NC, B, YW), jnp.float32)] * 3

    @functools.partial(
        pl.kernel, out_type=outs, mesh=_mesh(),
        compiler_params=_SC_PARAMS,
        scratch_types=[pltpu.VMEM((GC, YW), jnp.float32),
                       pltpu.VMEM((GC, YW), jnp.float32),
                       pltpu.VMEM((GC,), jnp.int32),
                       pltpu.VMEM((GC,), jnp.int32),
                       pltpu.SemaphoreType.DMA,
                       pltpu.SemaphoreType.DMA,
                       pltpu.VMEM_SHARED((B, YW), jnp.float32),
                       pltpu.VMEM_SHARED((B, YW), jnp.float32),
                       pltpu.VMEM_SHARED((B, YW), jnp.float32)])
    def k(yi, si, yu, su_, ys, ss, z, out_i, out_u, out_s,
          y_v, y_v1, seg_v, seg_v1, ysem0, ysem1, acc_i, acc_u, acc_s):
        c = lax.axis_index("c")
        s = lax.axis_index("s")
        w = s * NC + c

        @pl.when(s == 0)
        def _zero():
            pltpu.sync_copy(z, acc_i)
            pltpu.sync_copy(z, acc_u)
            pltpu.sync_copy(z, acc_s)

        plsc.subcore_barrier()

        def task(y, seg, acc, n):
            per = n // NW
            base = pl.multiple_of(w * per, 8)

            def body(i, carry):
                o0 = pl.multiple_of(base + (2 * i) * GC, 8)
                o1 = pl.multiple_of(base + (2 * i + 1) * GC, 8)
                h0 = pltpu.async_copy(y.at[pl.ds(o0, GC)], y_v, ysem0)
                h1 = pltpu.async_copy(y.at[pl.ds(o1, GC)], y_v1, ysem1)
                pltpu.sync_copy(seg.at[pl.ds(o0, GC)], seg_v)
                pltpu.sync_copy(seg.at[pl.ds(o1, GC)], seg_v1)
                h0.wait()
                pltpu.sync_copy(y_v, acc.at[seg_v], add=True)
                h1.wait()
                pltpu.sync_copy(y_v1, acc.at[seg_v1], add=True)
                return carry

            lax.fori_loop(0, per // (2 * GC), body, 0)

        task(yi, si, acc_i, E_ITEM)
        task(yu, su_, acc_u, E_ITEM)
        task(ys, ss, acc_s, E_SOC)

        plsc.subcore_barrier()
        rows = B // NS
        sl = pl.ds(s * rows, rows)
        pltpu.sync_copy(acc_i.at[sl], out_i.at[c, sl])
        pltpu.sync_copy(acc_u.at[sl], out_u.at[c, sl])
        pltpu.sync_copy(acc_s.at[sl], out_s.at[c, sl])

    return k(y_i, seg_i, y_u, seg_u, y_s, seg_s, zero_rows)


CI = 2048


def _tc_final(agg_i, agg_u, agg_s, item_table,
              eq4_w, eq4_b, eq4i_w, eq4i_b, eq9_w, eq9_b, eq13_w, eq13_b,
              mlp_w0, mlp_b0, mlp_w1, mlp_b1, mlp_w2, mlp_b2):
    grid = pl.cdiv(item_table.shape[0], CI)

    def body(ai, au, asoc, it, e4w, e4b, e4iw, e4ib, e9w, e9b,
             e13w, e13b, m0w, m0b, m1w, m1b, m2w, m2b,
             m_ref, sc_ref, h_scr):
        pid = pl.program_id(0)

        @pl.when(pid == 0)
        def _head():
            def head(agg, wt, bt):
                a = agg[0] + agg[1]
                v = a[:, :H]
                sden = a[:, H:]
                return jnp.maximum((v / (sden + 1e-16)) @ wt[...].T + bt[...],
                                   0.0)

            hi = head(ai[...], e4w, e4b)
            zj = head(au[...], e4iw, e4ib)
            hs = head(asoc[...], e9w, e9b)
            e13 = e13w[...]
            h = jnp.maximum(hi @ e13[:, :H].T + hs @ e13[:, H:].T + e13b[...],
                            0.0)
            h_scr[...] = h
            m0 = m0w[...]
            mm = h @ m0[:, :H].T + zj @ m0[:, H:].T + m0b[...]
            mm = jnp.maximum(mm, 0.0) @ m1w[...].T + m1b[...]
            mm = jnp.maximum(mm, 0.0) @ m2w[...].T + m2b[...]
            m_ref[...] = mm[:, :1]

        sc_ref[...] = lax.dot_general(it[:, :H], h_scr[...],
                                      (((1,), (1,)), ((), ())))

    full = lambda i: (0, 0)
    full3 = lambda i: (0, 0, 0)
    return pl.pallas_call(
        body,
        grid=(grid,),
        in_specs=[
            pl.BlockSpec((NC, B, YW), full3),
            pl.BlockSpec((NC, B, YW), full3),
            pl.BlockSpec((NC, B, YW), full3),
            pl.BlockSpec((CI, YW), lambda i: (i, 0)),
            pl.BlockSpec((H, H), full),
            pl.BlockSpec((1, H), full),
            pl.BlockSpec((H, H), full),
            pl.BlockSpec((1, H), full),
            pl.BlockSpec((H, H), full),
            pl.BlockSpec((1, H), full),
            pl.BlockSpec((H, 2 * H), full),
            pl.BlockSpec((1, H), full),
            pl.BlockSpec((H, 2 * H), full),
            pl.BlockSpec((1, H), full),
            pl.BlockSpec((H, H), full),
            pl.BlockSpec((1, H), full),
            pl.BlockSpec((H, H), full),
            pl.BlockSpec((1, 1), full),
        ],
        out_specs=[
            pl.BlockSpec((B, 1), lambda i: (0, 0)),
            pl.BlockSpec((CI, B), lambda i: (i, 0)),
        ],
        out_shape=[
            jax.ShapeDtypeStruct((B, 1), jnp.float32),
            jax.ShapeDtypeStruct((item_table.shape[0], B), jnp.float32),
        ],
        scratch_shapes=[pltpu.VMEM((B, H), jnp.float32)],
    )(agg_i, agg_u, agg_s, item_table,
      eq4_w, eq4_b, eq4i_w, eq4i_b, eq9_w, eq9_b, eq13_w, eq13_b,
      mlp_w0, mlp_b0, mlp_w1, mlp_b1, mlp_w2, mlp_b2)


def kernel(user, item, rating, item4user, social, social4user,
           batch_i_users, batch_i_ratings, batch_u_item, batch_target,
           user_table, item_table, rating_table,
           gv_w, gv_b, gu_w, gu_b,
           eq5_w0, eq5_b0, eq5_w1, eq5_b1,
           eq5i_w0, eq5i_b0, eq5i_w1, eq5i_b1,
           eq4_w, eq4_b, eq4i_w, eq4i_b,
           eq10_w0, eq10_b0, eq10_w1, eq10_b1,
           eq9_w, eq9_b, eq13_w, eq13_b,
           mlp_w0, mlp_b0, mlp_w1, mlp_b1, mlp_w2, mlp_b2):
    i32 = jnp.int32
    user = user.astype(i32)
    item = item.astype(i32)
    rating = rating.astype(i32)
    item4user = item4user.astype(i32)
    social = social.astype(i32)
    social4user = social4user.astype(i32)
    batch_i_users = batch_i_users.astype(i32)
    batch_i_ratings = batch_i_ratings.astype(i32)
    batch_u_item = batch_u_item.astype(i32)
    batch_target = batch_target.astype(i32)

    it128 = _tc_pad_table(item_table)
    ut128 = _tc_pad_table(user_table)
    tgt, item_emb = _sc_gather_item(it128, batch_target, item)
    user_emb, social_emb, item_u_emb = _sc_gather_user(
        ut128, user, social, batch_i_users)
    ue_g, su, tgt_g = _sc_gather_stage2(
        user_emb, tgt, item4user, social4user, batch_u_item)

    r2 = lambda b: b.reshape(1, -1)
    rep = lambda w: jnp.broadcast_to(w, (H, H))
    rids3 = rating.reshape(E_ITEM // CE, 1, CE)
    brids3 = batch_i_ratings.reshape(E_ITEM // CE, 1, CE)
    y_i = _tc_edges_proj(item_emb, rids3, ue_g, rating_table,
                         gv_w, r2(gv_b), eq5_w0, r2(eq5_b0), rep(eq5_w1),
                         r2(eq5_b1), E_ITEM)
    y_u = _tc_edges_proj(item_u_emb, brids3, tgt_g, rating_table,
                         gu_w, r2(gu_b), eq5i_w0, r2(eq5i_b0), rep(eq5i_w1),
                         r2(eq5i_b1), E_ITEM)
    y_s = _tc_edges_soc(social_emb, su, eq10_w0, r2(eq10_b0), rep(eq10_w1),
                        r2(eq10_b1))

    zero_rows = jnp.zeros((B, YW), jnp.float32)
    agg_i, agg_u, agg_s = _sc_segsum(y_i, item4user, y_u, batch_u_item,
                                     y_s, social4user, zero_rows)

    m, scores_t = _tc_final(agg_i, agg_u, agg_s, it128,
                            eq4_w, r2(eq4_b), eq4i_w, r2(eq4i_b),
                            eq9_w, r2(eq9_b), eq13_w, r2(eq13_b),
                            mlp_w0, r2(mlp_b0), mlp_w1, r2(mlp_b1),
                            rep(mlp_w2), mlp_b2.reshape(1, 1))
    return m, scores_t.T
